# Initial kernel scaffold; baseline (speedup 1.0000x reference)
import functools

import jax
import jax.numpy as jnp
from jax import lax
from jax.experimental import pallas as pl
from jax.experimental.pallas import tpu as pltpu
from jax.experimental.pallas import tpu_sc as plsc

N = 50000
D = 64
E = 800000
NUM_LAYERS = 3
ALPHA = 1.0 / (NUM_LAYERS + 1)

NC = 2
NS = 16
NPAD = 51200
TS = NPAD // NS
HALF = NPAD // NC
DUMP = HALF
ACC_ROWS = 26112
CH = 128
PAD_E = 802816
NCHUNK = PAD_E // CH
PROP_PT = NCHUNK // NS
DOT_PT = NCHUNK // (NC * NS)
PAD_COL = NPAD - 1

_i32 = jnp.int32
_f32 = jnp.float32


def _mesh():
    return plsc.VectorSubcoreMesh(
        core_axis_name="c", subcore_axis_name="s",
        num_cores=NC, num_subcores=NS)


def _rsqrt16(x):
    i = plsc.bitcast(x, _i32)
    i = jnp.int32(0x5F3759DF) - lax.shift_right_logical(i, 1)
    y = plsc.bitcast(i, _f32)
    for _ in range(3):
        y = y * (1.5 - 0.5 * x * y * y)
    return y


def _k1_body(col2d, x0, dis_out, y0_out, hist, stage, wrow, xbuf, cbuf):
    c = lax.axis_index("c")
    s = lax.axis_index("s")
    zeros16 = jnp.zeros((16,), _f32)
    ones16 = jnp.ones((16,), _f32)
    iota = lax.iota(_i32, 16)

    @pl.loop(0, NPAD, step=16)
    def _(j):
        hist[pl.ds(j, 16)] = zeros16

    @pl.loop(0, PROP_PT)
    def _(jj):
        chunk = s * PROP_PT + jj
        pltpu.sync_copy(col2d.at[pl.ds(chunk, 1)], cbuf)
        for g in range(CH // 16):
            cc = cbuf[0, pl.ds(g * 16, 16)]
            plsc.addupdate_scatter(hist, [cc], ones16)

    pltpu.sync_copy(hist, stage.at[s])
    plsc.subcore_barrier()

    base = s * TS
    @pl.loop(0, TS, step=16)
    def _(j):
        hist[pl.ds(j, 16)] = zeros16

    for w in range(NS):
        pltpu.sync_copy(stage.at[w, pl.ds(base, TS)], wrow)

        @pl.loop(0, TS, step=16)
        def _(j):
            hist[pl.ds(j, 16)] = hist[pl.ds(j, 16)] + wrow[pl.ds(j, 16)]

    @pl.loop(0, TS, step=16)
    def _(j):
        dv = hist[pl.ds(j, 16)]
        r = _rsqrt16(jnp.maximum(dv, 1.0))
        hist[pl.ds(j, 16)] = jnp.where(dv > 0, r, 0.0)

    @pl.when(c == 0)
    def _():
        pltpu.sync_copy(hist.at[pl.ds(0, TS)], dis_out.at[pl.ds(base, TS)])

    r0 = s * TS + c * (TS // NC)

    @pl.loop(0, TS // NC, step=16)
    def _(g):
        pltpu.sync_copy(x0.at[pl.ds(r0 + g, 16)], xbuf)
        dvec = hist[pl.ds(c * (TS // NC) + g, 16)]

        @pl.loop(0, D)
        def _(d):
            cv = jnp.full((16,), d, _i32)
            v = plsc.load_gather(xbuf, [iota, cv])
            plsc.store_scatter(xbuf, [iota, cv], v * dvec)

        pltpu.sync_copy(xbuf, y0_out.at[pl.ds(r0 + g, 16)])


def _layer_body(final, *refs):
    if final:
        (y_prev, dis, row2d, col2d, x0, x1, x2, x_out,
         acc, gbuf, ridx, lidx, cbuf, zbuf, wbuf, ybuf, dbuf,
         xb0, xb1, xb2) = refs
    else:
        (y_prev, dis, row2d, col2d, x_out, y_out,
         acc, gbuf, ridx, lidx, cbuf, zbuf, wbuf, ybuf, dbuf) = refs
    c = lax.axis_index("c")
    s = lax.axis_index("s")
    base = c * HALF
    zeros16 = jnp.zeros((16,), _f32)
    iota = lax.iota(_i32, 16)

    for r in range(16):
        for k in range(D // 16):
            zbuf[r, pl.ds(k * 16, 16)] = zeros16

    @pl.loop(0, ACC_ROWS // NS, step=16)
    def _(j):
        pltpu.sync_copy(zbuf, acc.at[pl.ds(s * (ACC_ROWS // NS) + j, 16)])

    plsc.subcore_barrier()

    @pl.loop(0, PROP_PT)
    def _(jj):
        chunk = s * PROP_PT + jj
        pltpu.sync_copy(row2d.at[chunk], ridx)
        pltpu.sync_copy(col2d.at[pl.ds(chunk, 1)], cbuf)
        for g in range(CH // 16):
            cc = cbuf[0, pl.ds(g * 16, 16)]
            ok = (cc >= base) & (cc < base + HALF)
            lidx[0, pl.ds(g * 16, 16)] = jnp.where(ok, cc - base, DUMP)
        pltpu.sync_copy(y_prev.at[ridx], gbuf)
        pltpu.sync_copy(gbuf, acc.at[lidx], add=True)

    plsc.subcore_barrier()

    r0l = s * (HALF // NS)

    @pl.loop(0, HALF // NS, step=16)
    def _(g):
        lr = r0l + g
        gr = base + lr
        pltpu.sync_copy(acc.at[pl.ds(lr, 16)], wbuf)
        pltpu.sync_copy(dis.at[pl.ds(gr, 16)], dbuf)
        if final:
            pltpu.sync_copy(x0.at[pl.ds(gr, 16)], xb0)
            pltpu.sync_copy(x1.at[pl.ds(gr, 16)], xb1)
            pltpu.sync_copy(x2.at[pl.ds(gr, 16)], xb2)
        dvec = dbuf[...]

        @pl.loop(0, D)
        def _(d):
            cv = jnp.full((16,), d, _i32)
            xv = plsc.load_gather(wbuf, [iota, cv]) * dvec
            if final:
                v0 = plsc.load_gather(xb0, [iota, cv])
                v1 = plsc.load_gather(xb1, [iota, cv])
                v2 = plsc.load_gather(xb2, [iota, cv])
                plsc.store_scatter(wbuf, [iota, cv],
                                   (v0 + v1 + v2 + xv) * ALPHA)
            else:
                plsc.store_scatter(wbuf, [iota, cv], xv)
                plsc.store_scatter(ybuf, [iota, cv], xv * dvec)

        pltpu.sync_copy(wbuf, x_out.at[pl.ds(gr, 16)])
        if not final:
            pltpu.sync_copy(ybuf, y_out.at[pl.ds(gr, 16)])


def _dot_body(out_t, row2d, col2d, res2d, abuf, bbuf, ridx, cidx, rbuf):
    c = lax.axis_index("c")
    s = lax.axis_index("s")
    wid = s * NC + c
    iota = lax.iota(_i32, 16)
    zeros16 = jnp.zeros((16,), _f32)

    @pl.loop(0, DOT_PT)
    def _(jj):
        chunk = wid * DOT_PT + jj
        pltpu.sync_copy(row2d.at[chunk], ridx)
        pltpu.sync_copy(col2d.at[chunk], cidx)
        pltpu.sync_copy(out_t.at[ridx], abuf)
        pltpu.sync_copy(out_t.at[cidx], bbuf)
        for g in range(CH // 16):
            rbuf[0, pl.ds(g * 16, 16)] = zeros16

        @pl.loop(0, D)
        def _(d):
            cv = jnp.full((16,), d, _i32)
            for g in range(CH // 16):
                rv = iota + g * 16
                a = plsc.load_gather(abuf, [rv, cv])
                b = plsc.load_gather(bbuf, [rv, cv])
                plsc.addupdate(rbuf.at[0, pl.ds(g * 16, 16)], a * b)

        pltpu.sync_copy(rbuf, res2d.at[pl.ds(chunk, 1)])


def kernel(edge_index, embedding_weight):
    row = edge_index[0]
    col = edge_index[1]
    pad = PAD_E - E
    row2d = jnp.pad(row, (0, pad)).reshape(NCHUNK, CH)
    col2d = jnp.pad(col, (0, pad), constant_values=PAD_COL).reshape(
        NCHUNK, CH)
    x0 = jnp.pad(embedding_weight, ((0, NPAD - N), (0, 0)))

    nd = jax.ShapeDtypeStruct((NPAD, D), _f32)

    k1 = pl.kernel(
        _k1_body,
        out_type=[jax.ShapeDtypeStruct((NPAD,), _f32), nd],
        mesh=_mesh(),
        scratch_types=[
            pltpu.VMEM((NPAD,), _f32),
            pltpu.VMEM_SHARED((NS, NPAD), _f32),
            pltpu.VMEM((TS,), _f32),
            pltpu.VMEM((16, D), _f32),
            pltpu.VMEM((1, CH), _i32),
        ])
    dis, y0 = k1(col2d, x0)

    layer_scratch = [
        pltpu.VMEM_SHARED((ACC_ROWS, D), _f32),
        pltpu.VMEM((CH, D), _f32),
        pltpu.VMEM((CH,), _i32),
        pltpu.VMEM((1, CH), _i32),
        pltpu.VMEM((1, CH), _i32),
        pltpu.VMEM((16, D), _f32),
        pltpu.VMEM((16, D), _f32),
        pltpu.VMEM((16, D), _f32),
        pltpu.VMEM((16,), _f32),
    ]
    klayer = pl.kernel(
        functools.partial(_layer_body, False),
        out_type=[nd, nd],
        mesh=_mesh(),
        scratch_types=layer_scratch)
    x1, y1 = klayer(y0, dis, row2d, col2d)
    x2, y2 = klayer(y1, dis, row2d, col2d)

    kfinal = pl.kernel(
        functools.partial(_layer_body, True),
        out_type=[nd],
        mesh=_mesh(),
        scratch_types=layer_scratch + [
            pltpu.VMEM((16, D), _f32),
            pltpu.VMEM((16, D), _f32),
            pltpu.VMEM((16, D), _f32),
        ])
    out_t = kfinal(y2, dis, row2d, col2d, x0, x1, x2)

    kdot = pl.kernel(
        _dot_body,
        out_type=[jax.ShapeDtypeStruct((NCHUNK, CH), _f32)],
        mesh=_mesh(),
        scratch_types=[
            pltpu.VMEM((CH, D), _f32),
            pltpu.VMEM((CH, D), _f32),
            pltpu.VMEM((CH,), _i32),
            pltpu.VMEM((CH,), _i32),
            pltpu.VMEM((1, CH), _f32),
        ])
    res2d = kdot(out_t, row2d, col2d)
    return res2d.reshape(-1)[:E]


# speedup vs baseline: 3.6145x; 3.6145x over previous
import functools

import jax
import jax.numpy as jnp
from jax import lax
from jax.experimental import pallas as pl
from jax.experimental.pallas import tpu as pltpu
from jax.experimental.pallas import tpu_sc as plsc

N = 50000
D = 64
E = 800000
NUM_LAYERS = 3
ALPHA = 1.0 / (NUM_LAYERS + 1)

NC = 2
NS = 16
NPAD = 51200
TS = NPAD // NS
HALF = NPAD // NC
DUMP = HALF
ACC_ROWS = 26112
CH = 128
PAD_E = 802816
NCHUNK = PAD_E // CH
PROP_PT = NCHUNK // NS
DOT_PT = NCHUNK // (NC * NS)
PAD_COL = NPAD - 1

_i32 = jnp.int32
_f32 = jnp.float32


_CP = pltpu.CompilerParams(needs_layout_passes=False,
                           use_tc_tiling_on_sc=False)


def _mesh():
    return plsc.VectorSubcoreMesh(
        core_axis_name="c", subcore_axis_name="s",
        num_cores=NC, num_subcores=NS)


def _rsqrt16(x):
    i = plsc.bitcast(x, _i32)
    i = jnp.int32(0x5F3759DF) - lax.shift_right_logical(i, 1)
    y = plsc.bitcast(i, _f32)
    for _ in range(3):
        y = y * (1.5 - 0.5 * x * y * y)
    return y


def _k1_body(col2d, x0, dis_out, y0_out, hist, stage, wrow, xbuf, cbuf):
    c = lax.axis_index("c")
    s = lax.axis_index("s")
    zeros16 = jnp.zeros((16,), _f32)
    ones16 = jnp.ones((16,), _f32)
    iota = lax.iota(_i32, 16)

    @pl.loop(0, NPAD, step=16)
    def _(j):
        hist[pl.ds(j, 16)] = zeros16

    @pl.loop(0, PROP_PT)
    def _(jj):
        chunk = s * PROP_PT + jj
        pltpu.sync_copy(col2d.at[chunk], cbuf)
        for g in range(CH // 16):
            cc = cbuf[pl.ds(g * 16, 16)]
            plsc.addupdate_scatter(hist, [cc], ones16)

    pltpu.sync_copy(hist, stage.at[s])
    plsc.subcore_barrier()

    base = s * TS
    @pl.loop(0, TS, step=16)
    def _(j):
        hist[pl.ds(j, 16)] = zeros16

    for w in range(NS):
        pltpu.sync_copy(stage.at[w, pl.ds(base, TS)], wrow)

        @pl.loop(0, TS, step=16)
        def _(j):
            hist[pl.ds(j, 16)] = hist[pl.ds(j, 16)] + wrow[pl.ds(j, 16)]

    @pl.loop(0, TS, step=16)
    def _(j):
        dv = hist[pl.ds(j, 16)]
        r = _rsqrt16(jnp.maximum(dv, 1.0))
        hist[pl.ds(j, 16)] = jnp.where(dv > 0, r, 0.0)

    @pl.when(c == 0)
    def _():
        pltpu.sync_copy(hist.at[pl.ds(0, TS)], dis_out.at[pl.ds(base, TS)])

    r0 = s * TS + c * (TS // NC)

    @pl.loop(0, TS // NC, step=16)
    def _(g):
        pltpu.sync_copy(x0.at[pl.ds(r0 + g, 16)], xbuf)
        dvec = hist[pl.ds(c * (TS // NC) + g, 16)]

        @pl.loop(0, D)
        def _(d):
            cv = jnp.full((16,), d, _i32)
            v = plsc.load_gather(xbuf, [iota, cv])
            plsc.store_scatter(xbuf, [iota, cv], v * dvec)

        pltpu.sync_copy(xbuf, y0_out.at[pl.ds(r0 + g, 16)])


def _layer_body(final, *refs):
    if final:
        (y_prev, dis, row2d, col2d, x0, x1, x2, x_out,
         acc, gbuf, ridx, lidx, cbuf, zbuf, wbuf, ybuf, dbuf,
         xb0, xb1, xb2) = refs
    else:
        (y_prev, dis, row2d, col2d, x_out, y_out,
         acc, gbuf, ridx, lidx, cbuf, zbuf, wbuf, ybuf, dbuf) = refs
    c = lax.axis_index("c")
    s = lax.axis_index("s")
    base = c * HALF
    zeros16 = jnp.zeros((16,), _f32)
    iota = lax.iota(_i32, 16)

    for r in range(16):
        for k in range(D // 16):
            zbuf[r, pl.ds(k * 16, 16)] = zeros16

    @pl.loop(0, ACC_ROWS // NS, step=16)
    def _(j):
        pltpu.sync_copy(zbuf, acc.at[pl.ds(s * (ACC_ROWS // NS) + j, 16)])

    plsc.subcore_barrier()

    @pl.loop(0, PROP_PT)
    def _(jj):
        chunk = s * PROP_PT + jj
        pltpu.sync_copy(row2d.at[chunk], ridx)
        pltpu.sync_copy(col2d.at[chunk], cbuf)
        for g in range(CH // 16):
            cc = cbuf[pl.ds(g * 16, 16)]
            ok = (cc >= base) & (cc < base + HALF)
            lidx[pl.ds(g * 16, 16)] = jnp.where(ok, cc - base, DUMP)
        pltpu.sync_copy(y_prev.at[ridx], gbuf)
        pltpu.sync_copy(gbuf, acc.at[lidx], add=True)

    plsc.subcore_barrier()

    r0l = s * (HALF // NS)

    @pl.loop(0, HALF // NS, step=16)
    def _(g):
        lr = r0l + g
        gr = base + lr
        pltpu.sync_copy(acc.at[pl.ds(lr, 16)], wbuf)
        pltpu.sync_copy(dis.at[pl.ds(gr, 16)], dbuf)
        if final:
            pltpu.sync_copy(x0.at[pl.ds(gr, 16)], xb0)
            pltpu.sync_copy(x1.at[pl.ds(gr, 16)], xb1)
            pltpu.sync_copy(x2.at[pl.ds(gr, 16)], xb2)
        dvec = dbuf[...]

        @pl.loop(0, D)
        def _(d):
            cv = jnp.full((16,), d, _i32)
            xv = plsc.load_gather(wbuf, [iota, cv]) * dvec
            if final:
                v0 = plsc.load_gather(xb0, [iota, cv])
                v1 = plsc.load_gather(xb1, [iota, cv])
                v2 = plsc.load_gather(xb2, [iota, cv])
                plsc.store_scatter(wbuf, [iota, cv],
                                   (v0 + v1 + v2 + xv) * ALPHA)
            else:
                plsc.store_scatter(wbuf, [iota, cv], xv)
                plsc.store_scatter(ybuf, [iota, cv], xv * dvec)

        pltpu.sync_copy(wbuf, x_out.at[pl.ds(gr, 16)])
        if not final:
            pltpu.sync_copy(ybuf, y_out.at[pl.ds(gr, 16)])


def _dot_body(out_t, row2d, col2d, res2d, abuf, bbuf, ridx, cidx, rbuf):
    c = lax.axis_index("c")
    s = lax.axis_index("s")
    wid = s * NC + c
    iota = lax.iota(_i32, 16)
    zeros16 = jnp.zeros((16,), _f32)

    @pl.loop(0, DOT_PT)
    def _(jj):
        chunk = wid * DOT_PT + jj
        pltpu.sync_copy(row2d.at[chunk], ridx)
        pltpu.sync_copy(col2d.at[chunk], cidx)
        pltpu.sync_copy(out_t.at[ridx], abuf)
        pltpu.sync_copy(out_t.at[cidx], bbuf)
        for g in range(CH // 16):
            rbuf[pl.ds(g * 16, 16)] = zeros16

        @pl.loop(0, D)
        def _(d):
            cv = jnp.full((16,), d, _i32)
            for g in range(CH // 16):
                rv = iota + g * 16
                a = plsc.load_gather(abuf, [rv, cv])
                b = plsc.load_gather(bbuf, [rv, cv])
                plsc.addupdate(rbuf.at[pl.ds(g * 16, 16)], a * b)

        pltpu.sync_copy(rbuf, res2d.at[chunk])


def kernel(edge_index, embedding_weight):
    row = edge_index[0]
    col = edge_index[1]
    pad = PAD_E - E
    row2d = jnp.pad(row, (0, pad)).reshape(NCHUNK, CH)
    col2d = jnp.pad(col, (0, pad), constant_values=PAD_COL).reshape(
        NCHUNK, CH)
    x0 = jnp.pad(embedding_weight, ((0, NPAD - N), (0, 0)))

    nd = jax.ShapeDtypeStruct((NPAD, D), _f32)

    k1 = pl.kernel(
        _k1_body,
        out_type=[jax.ShapeDtypeStruct((NPAD,), _f32), nd],
        mesh=_mesh(),
        compiler_params=_CP,
        scratch_types=[
            pltpu.VMEM((NPAD,), _f32),
            pltpu.VMEM_SHARED((NS, NPAD), _f32),
            pltpu.VMEM((TS,), _f32),
            pltpu.VMEM((16, D), _f32),
            pltpu.VMEM((CH,), _i32),
        ])
    dis, y0 = k1(col2d, x0)

    layer_scratch = [
        pltpu.VMEM_SHARED((ACC_ROWS, D), _f32),
        pltpu.VMEM((CH, D), _f32),
        pltpu.VMEM((CH,), _i32),
        pltpu.VMEM((CH,), _i32),
        pltpu.VMEM((CH,), _i32),
        pltpu.VMEM((16, D), _f32),
        pltpu.VMEM((16, D), _f32),
        pltpu.VMEM((16, D), _f32),
        pltpu.VMEM((16,), _f32),
    ]
    klayer = pl.kernel(
        functools.partial(_layer_body, False),
        out_type=[nd, nd],
        mesh=_mesh(),
        compiler_params=_CP,
        scratch_types=layer_scratch)
    x1, y1 = klayer(y0, dis, row2d, col2d)
    x2, y2 = klayer(y1, dis, row2d, col2d)

    kfinal = pl.kernel(
        functools.partial(_layer_body, True),
        out_type=[nd],
        mesh=_mesh(),
        compiler_params=_CP,
        scratch_types=layer_scratch + [
            pltpu.VMEM((16, D), _f32),
            pltpu.VMEM((16, D), _f32),
            pltpu.VMEM((16, D), _f32),
        ])
    (out_t,) = kfinal(y2, dis, row2d, col2d, x0, x1, x2)

    kdot = pl.kernel(
        _dot_body,
        out_type=[jax.ShapeDtypeStruct((NCHUNK, CH), _f32)],
        mesh=_mesh(),
        compiler_params=_CP,
        scratch_types=[
            pltpu.VMEM((CH, D), _f32),
            pltpu.VMEM((CH, D), _f32),
            pltpu.VMEM((CH,), _i32),
            pltpu.VMEM((CH,), _i32),
            pltpu.VMEM((CH,), _f32),
        ])
    (res2d,) = kdot(out_t, row2d, col2d)
    return res2d.reshape(-1)[:E]


# speedup vs baseline: 4.7094x; 1.3029x over previous
import functools

import jax
import jax.numpy as jnp
from jax import lax
from jax.experimental import pallas as pl
from jax.experimental.pallas import tpu as pltpu
from jax.experimental.pallas import tpu_sc as plsc

N = 50000
D = 64
E = 800000
NUM_LAYERS = 3
ALPHA = 1.0 / (NUM_LAYERS + 1)

NC = 2
NS = 16
NPAD = 51200
TS = NPAD // NS
HALF = NPAD // NC
DUMP = HALF
ACC_ROWS = 25920
CHL = 256
CHD = 512
RCH = 80
RCB = 320
PAD_E = 802816
NCHUNK_L = PAD_E // CHL
NCHUNK_D = PAD_E // CHD
PROP_PT = NCHUNK_L // NS
DOT_PT = NCHUNK_D // (NC * NS)
PAD_COL = NPAD - 1

_i32 = jnp.int32
_f32 = jnp.float32

_CP = pltpu.CompilerParams(needs_layout_passes=False,
                           use_tc_tiling_on_sc=False)


def _mesh():
    return plsc.VectorSubcoreMesh(
        core_axis_name="c", subcore_axis_name="s",
        num_cores=NC, num_subcores=NS)


def _rsqrt16(x):
    i = plsc.bitcast(x, _i32)
    i = jnp.int32(0x5F3759DF) - lax.shift_right_logical(i, 1)
    y = plsc.bitcast(i, _f32)
    for _ in range(3):
        y = y * (1.5 - 0.5 * x * y * y)
    return y


def _k1_body(col2d, x0, dis_out, y0_out, hist, stage, wrow, xbuf, cbuf):
    c = lax.axis_index("c")
    s = lax.axis_index("s")
    zeros16 = jnp.zeros((16,), _f32)
    ones16 = jnp.ones((16,), _f32)
    iota = lax.iota(_i32, 16)

    @pl.loop(0, NPAD, step=16)
    def _(j):
        hist[pl.ds(j, 16)] = zeros16

    @pl.loop(0, PROP_PT)
    def _(jj):
        chunk = s * PROP_PT + jj
        pltpu.sync_copy(col2d.at[chunk], cbuf)
        for g in range(CHL // 16):
            cc = cbuf[pl.ds(g * 16, 16)]
            plsc.addupdate_scatter(hist, [cc], ones16)

    pltpu.sync_copy(hist, stage.at[s])
    plsc.subcore_barrier()

    base = s * TS
    @pl.loop(0, TS, step=16)
    def _(j):
        hist[pl.ds(j, 16)] = zeros16

    for w in range(NS):
        pltpu.sync_copy(stage.at[w, pl.ds(base, TS)], wrow)

        @pl.loop(0, TS, step=16)
        def _(j):
            hist[pl.ds(j, 16)] = hist[pl.ds(j, 16)] + wrow[pl.ds(j, 16)]

    @pl.loop(0, TS, step=16)
    def _(j):
        dv = hist[pl.ds(j, 16)]
        r = _rsqrt16(jnp.maximum(dv, 1.0))
        hist[pl.ds(j, 16)] = jnp.where(dv > 0, r, 0.0)

    @pl.when(c == 0)
    def _():
        pltpu.sync_copy(hist.at[pl.ds(0, TS)], dis_out.at[pl.ds(base, TS)])

    r0 = s * TS + c * (TS // NC)

    @pl.loop(0, TS // NC, step=RCH)
    def _(g):
        pltpu.sync_copy(x0.at[pl.ds(r0 + g, RCH)], xbuf)

        @pl.loop(0, RCH, step=16)
        def _(rr):
            dvec = hist[pl.ds(c * (TS // NC) + g + rr, 16)]
            rv = iota + rr

            @pl.loop(0, D)
            def _(d):
                cv = jnp.full((16,), d, _i32)
                v = plsc.load_gather(xbuf, [rv, cv])
                plsc.store_scatter(xbuf, [rv, cv], v * dvec)

        pltpu.sync_copy(xbuf, y0_out.at[pl.ds(r0 + g, RCH)])


def _layer_body(final, y_prev, dis, row2d, col2d, x_out, y_out,
                acc, gbuf, ridx, lidx, cbuf, ybuf, dbuf, semg, sems):
    c = lax.axis_index("c")
    s = lax.axis_index("s")
    base = c * HALF
    zeros16 = jnp.zeros((16,), _f32)
    iota = lax.iota(_i32, 16)

    @pl.loop(0, 162)
    def _(r):
        for k in range(D // 16):
            gbuf[r, pl.ds(k * 16, 16)] = zeros16

    @pl.loop(0, ACC_ROWS // NS, step=162)
    def _(j):
        pltpu.sync_copy(gbuf.at[pl.ds(0, 162)],
                        acc.at[pl.ds(s * (ACC_ROWS // NS) + j, 162)])

    plsc.subcore_barrier()

    @pl.loop(0, PROP_PT)
    def _(jj):
        chunk = s * PROP_PT + jj
        pltpu.sync_copy(row2d.at[chunk], ridx)
        pltpu.sync_copy(col2d.at[chunk], cbuf)
        for g in range(CHL // 16):
            cc = cbuf[pl.ds(g * 16, 16)]
            ok = (cc >= base) & (cc < base + HALF)
            lidx[g // 8, pl.ds((g % 8) * 16, 16)] = jnp.where(
                ok, cc - base, DUMP)

        @pl.loop(0, CHL // 128)
        def _(k):
            pltpu.async_copy(y_prev.at[ridx.at[pl.ds(k * 128, 128)]],
                             gbuf.at[pl.ds(k * 128, 128)], semg)

        @pl.loop(0, CHL // 128)
        def _(k):
            pltpu.make_async_copy(
                y_prev.at[ridx.at[pl.ds(k * 128, 128)]],
                gbuf.at[pl.ds(k * 128, 128)], semg).wait()

        @pl.loop(0, CHL // 128)
        def _(k):
            pltpu.async_copy(gbuf.at[pl.ds(k * 128, 128)],
                             acc.at[lidx.at[k]], sems, add=True)

        @pl.loop(0, CHL // 128)
        def _(k):
            pltpu.make_async_copy(gbuf.at[pl.ds(k * 128, 128)],
                                  acc.at[lidx.at[k]], sems).wait()

    plsc.subcore_barrier()

    r0l = s * (HALF // NS)

    @pl.loop(0, HALF // NS, step=RCH)
    def _(g):
        lr = r0l + g
        gr = base + lr
        wbuf = gbuf.at[pl.ds(0, RCH)]
        pltpu.sync_copy(acc.at[pl.ds(lr, RCH)], wbuf)
        pltpu.sync_copy(dis.at[pl.ds(gr, RCH)], dbuf)

        @pl.loop(0, RCH, step=16)
        def _(rr):
            dvec = dbuf[pl.ds(rr, 16)]
            rv = iota + rr

            @pl.loop(0, D)
            def _(d):
                cv = jnp.full((16,), d, _i32)
                xv = plsc.load_gather(wbuf, [rv, cv]) * dvec
                plsc.store_scatter(wbuf, [rv, cv], xv)
                if not final:
                    plsc.store_scatter(ybuf, [rv, cv], xv * dvec)

        pltpu.sync_copy(wbuf, x_out.at[pl.ds(gr, RCH)])
        if not final:
            pltpu.sync_copy(ybuf, y_out.at[pl.ds(gr, RCH)])


def _combine_body(x0, x1, x2, x3, out, b0, b1, b2, b3):
    c = lax.axis_index("c")
    s = lax.axis_index("s")
    r0 = s * TS + c * (TS // NC)

    @pl.loop(0, TS // NC, step=RCB)
    def _(g):
        pltpu.sync_copy(x0.at[pl.ds(r0 + g, RCB)], b0)
        pltpu.sync_copy(x1.at[pl.ds(r0 + g, RCB)], b1)
        pltpu.sync_copy(x2.at[pl.ds(r0 + g, RCB)], b2)
        pltpu.sync_copy(x3.at[pl.ds(r0 + g, RCB)], b3)

        @pl.loop(0, RCB)
        def _(r):
            for k in range(D // 16):
                sl = pl.ds(k * 16, 16)
                b0[r, sl] = (b0[r, sl] + b1[r, sl]
                             + b2[r, sl] + b3[r, sl]) * ALPHA

        pltpu.sync_copy(b0, out.at[pl.ds(r0 + g, RCB)])


def _dot_body(out_t, row2d, col2d, res2d, abuf, bbuf, ridx, cidx, rbuf):
    c = lax.axis_index("c")
    s = lax.axis_index("s")
    wid = s * NC + c
    iota = lax.iota(_i32, 16)
    zeros16 = jnp.zeros((16,), _f32)

    @pl.loop(0, DOT_PT)
    def _(jj):
        chunk = wid * DOT_PT + jj
        pltpu.sync_copy(row2d.at[chunk], ridx)
        pltpu.sync_copy(col2d.at[chunk], cidx)
        pltpu.sync_copy(out_t.at[ridx], abuf)
        pltpu.sync_copy(out_t.at[cidx], bbuf)
        for g in range(CHD // 16):
            rbuf[pl.ds(g * 16, 16)] = zeros16

        @pl.loop(0, D)
        def _(d):
            cv = jnp.full((16,), d, _i32)
            for g in range(CHD // 16):
                rv = iota + g * 16
                a = plsc.load_gather(abuf, [rv, cv])
                b = plsc.load_gather(bbuf, [rv, cv])
                plsc.addupdate(rbuf.at[pl.ds(g * 16, 16)], a * b)

        pltpu.sync_copy(rbuf, res2d.at[chunk])


def kernel(edge_index, embedding_weight):
    row = edge_index[0]
    col = edge_index[1]
    pad = PAD_E - E
    rowp = jnp.pad(row, (0, pad))
    colp = jnp.pad(col, (0, pad), constant_values=PAD_COL)
    row2dl = rowp.reshape(NCHUNK_L, CHL)
    col2dl = colp.reshape(NCHUNK_L, CHL)
    row2dd = rowp.reshape(NCHUNK_D, CHD)
    col2dd = colp.reshape(NCHUNK_D, CHD)
    x0 = jnp.pad(embedding_weight, ((0, NPAD - N), (0, 0)))

    nd = jax.ShapeDtypeStruct((NPAD, D), _f32)

    k1 = pl.kernel(
        _k1_body,
        out_type=[jax.ShapeDtypeStruct((NPAD,), _f32), nd],
        mesh=_mesh(),
        compiler_params=_CP,
        scratch_types=[
            pltpu.VMEM((NPAD,), _f32),
            pltpu.VMEM_SHARED((NS, NPAD), _f32),
            pltpu.VMEM((TS,), _f32),
            pltpu.VMEM((RCH, D), _f32),
            pltpu.VMEM((CHL,), _i32),
        ])
    dis, y0 = k1(col2dl, x0)

    layer_scratch = [
        pltpu.VMEM_SHARED((ACC_ROWS, D), _f32),
        pltpu.VMEM((CHL, D), _f32),
        pltpu.VMEM((CHL,), _i32),
        pltpu.VMEM((CHL // 128, 128), _i32),
        pltpu.VMEM((CHL,), _i32),
        pltpu.VMEM((RCH, D), _f32),
        pltpu.VMEM((RCH,), _f32),
        pltpu.SemaphoreType.DMA,
        pltpu.SemaphoreType.DMA,
    ]
    klayer = pl.kernel(
        functools.partial(_layer_body, False),
        out_type=[nd, nd],
        mesh=_mesh(),
        compiler_params=_CP,
        scratch_types=layer_scratch)
    x1, y1 = klayer(y0, dis, row2dl, col2dl)
    x2, y2 = klayer(y1, dis, row2dl, col2dl)

    kfinal = pl.kernel(
        functools.partial(_layer_body, True),
        out_type=[nd, nd],
        mesh=_mesh(),
        compiler_params=_CP,
        scratch_types=layer_scratch)
    x3, _y3 = kfinal(y2, dis, row2dl, col2dl)

    kcombine = pl.kernel(
        _combine_body,
        out_type=[nd],
        mesh=_mesh(),
        compiler_params=_CP,
        scratch_types=[
            pltpu.VMEM((RCB, D), _f32),
            pltpu.VMEM((RCB, D), _f32),
            pltpu.VMEM((RCB, D), _f32),
            pltpu.VMEM((RCB, D), _f32),
        ])
    (out_t,) = kcombine(x0, x1, x2, x3)

    kdot = pl.kernel(
        _dot_body,
        out_type=[jax.ShapeDtypeStruct((NCHUNK_D, CHD), _f32)],
        mesh=_mesh(),
        compiler_params=_CP,
        scratch_types=[
            pltpu.VMEM((CHD, D), _f32),
            pltpu.VMEM((CHD, D), _f32),
            pltpu.VMEM((CHD,), _i32),
            pltpu.VMEM((CHD,), _i32),
            pltpu.VMEM((CHD,), _f32),
        ])
    (res2d,) = kdot(out_t, row2dd, col2dd)
    return res2d.reshape(-1)[:E]


# speedup vs baseline: 8.3908x; 1.7817x over previous
import functools

import jax
import jax.numpy as jnp
from jax import lax
from jax.experimental import pallas as pl
from jax.experimental.pallas import tpu as pltpu
from jax.experimental.pallas import tpu_sc as plsc

N = 50000
D = 64
E = 800000
NUM_LAYERS = 3
ALPHA = 1.0 / (NUM_LAYERS + 1)

NC = 2
NS = 16
NPAD = 51200
TS = NPAD // NS
HALF = NPAD // NC
DUMP = HALF
ACC_ROWS = 25920
CHL = 256
CHD = 512
RCH = 80
RCB = 320
PAD_E = 802816
NCHUNK_L = PAD_E // CHL
NCHUNK_D = PAD_E // CHD
PROP_PT = NCHUNK_L // NS
DOT_PT = NCHUNK_D // (NC * NS)
PAD_COL = NPAD - 1

_i32 = jnp.int32
_f32 = jnp.float32

_CP = pltpu.CompilerParams(needs_layout_passes=False,
                           use_tc_tiling_on_sc=False)


def _mesh():
    return plsc.VectorSubcoreMesh(
        core_axis_name="c", subcore_axis_name="s",
        num_cores=NC, num_subcores=NS)


def _rsqrt16(x):
    i = plsc.bitcast(x, _i32)
    i = jnp.int32(0x5F3759DF) - lax.shift_right_logical(i, 1)
    y = plsc.bitcast(i, _f32)
    for _ in range(3):
        y = y * (1.5 - 0.5 * x * y * y)
    return y


def _k1_body(col2d, x0, dis_out, y0_out, hist, stage, wrow, xbuf, cbuf):
    c = lax.axis_index("c")
    s = lax.axis_index("s")
    zeros16 = jnp.zeros((16,), _f32)
    ones16 = jnp.ones((16,), _f32)
    iota = lax.iota(_i32, 16)

    @pl.loop(0, NPAD, step=16)
    def _(j):
        hist[pl.ds(j, 16)] = zeros16

    @pl.loop(0, PROP_PT)
    def _(jj):
        chunk = s * PROP_PT + jj
        pltpu.sync_copy(col2d.at[chunk], cbuf)
        for g in range(CHL // 16):
            cc = cbuf[pl.ds(g * 16, 16)]
            plsc.addupdate_scatter(hist, [cc], ones16)

    pltpu.sync_copy(hist, stage.at[s])
    plsc.subcore_barrier()

    base = s * TS
    @pl.loop(0, TS, step=16)
    def _(j):
        hist[pl.ds(j, 16)] = zeros16

    for w in range(NS):
        pltpu.sync_copy(stage.at[w, pl.ds(base, TS)], wrow)

        @pl.loop(0, TS, step=16)
        def _(j):
            hist[pl.ds(j, 16)] = hist[pl.ds(j, 16)] + wrow[pl.ds(j, 16)]

    @pl.loop(0, TS, step=16)
    def _(j):
        dv = hist[pl.ds(j, 16)]
        r = _rsqrt16(jnp.maximum(dv, 1.0))
        hist[pl.ds(j, 16)] = jnp.where(dv > 0, r, 0.0)

    @pl.when(c == 0)
    def _():
        pltpu.sync_copy(hist.at[pl.ds(0, TS)], dis_out.at[pl.ds(base, TS)])

    r0 = s * TS + c * (TS // NC)

    @pl.loop(0, TS // NC, step=RCH)
    def _(g):
        pltpu.sync_copy(x0.at[pl.ds(r0 + g, RCH)], xbuf)

        @pl.loop(0, RCH, step=16)
        def _(rr):
            dvec = hist[pl.ds(c * (TS // NC) + g + rr, 16)]
            for e in range(16):
                dv = dvec[jnp.full((16,), e, _i32)]
                r = rr + e
                for k in range(D // 16):
                    sl = pl.ds(k * 16, 16)
                    xbuf[r, sl] = xbuf[r, sl] * dv

        pltpu.sync_copy(xbuf, y0_out.at[pl.ds(r0 + g, RCH)])


def _layer_body(final, y_prev, dis, row2d, col2d, x_out, y_out,
                acc, gbuf, ridx, lidx, cbuf, ybuf, dbuf, semg, sems):
    c = lax.axis_index("c")
    s = lax.axis_index("s")
    base = c * HALF
    zeros16 = jnp.zeros((16,), _f32)
    iota = lax.iota(_i32, 16)

    @pl.loop(0, 162)
    def _(r):
        for k in range(D // 16):
            gbuf[r, pl.ds(k * 16, 16)] = zeros16

    @pl.loop(0, ACC_ROWS // NS, step=162)
    def _(j):
        pltpu.sync_copy(gbuf.at[pl.ds(0, 162)],
                        acc.at[pl.ds(s * (ACC_ROWS // NS) + j, 162)])

    plsc.subcore_barrier()

    @pl.loop(0, PROP_PT)
    def _(jj):
        chunk = s * PROP_PT + jj
        pltpu.sync_copy(row2d.at[chunk], ridx)
        pltpu.sync_copy(col2d.at[chunk], cbuf)
        for g in range(CHL // 16):
            cc = cbuf[pl.ds(g * 16, 16)]
            ok = (cc >= base) & (cc < base + HALF)
            lidx[g // 8, pl.ds((g % 8) * 16, 16)] = jnp.where(
                ok, cc - base, DUMP)

        @pl.loop(0, CHL // 128)
        def _(k):
            pltpu.async_copy(y_prev.at[ridx.at[pl.ds(k * 128, 128)]],
                             gbuf.at[pl.ds(k * 128, 128)], semg)

        @pl.loop(0, CHL // 128)
        def _(k):
            pltpu.make_async_copy(
                y_prev.at[ridx.at[pl.ds(k * 128, 128)]],
                gbuf.at[pl.ds(k * 128, 128)], semg).wait()

        @pl.loop(0, CHL // 128)
        def _(k):
            pltpu.async_copy(gbuf.at[pl.ds(k * 128, 128)],
                             acc.at[lidx.at[k]], sems, add=True)

        @pl.loop(0, CHL // 128)
        def _(k):
            pltpu.make_async_copy(gbuf.at[pl.ds(k * 128, 128)],
                                  acc.at[lidx.at[k]], sems).wait()

    plsc.subcore_barrier()

    r0l = s * (HALF // NS)

    @pl.loop(0, HALF // NS, step=RCH)
    def _(g):
        lr = r0l + g
        gr = base + lr
        wbuf = gbuf.at[pl.ds(0, RCH)]
        pltpu.sync_copy(acc.at[pl.ds(lr, RCH)], wbuf)
        pltpu.sync_copy(dis.at[pl.ds(gr, RCH)], dbuf)

        @pl.loop(0, RCH, step=16)
        def _(rr):
            dvec = dbuf[pl.ds(rr, 16)]
            for e in range(16):
                dv = dvec[jnp.full((16,), e, _i32)]
                r = rr + e
                for k in range(D // 16):
                    sl = pl.ds(k * 16, 16)
                    xv = wbuf[r, sl] * dv
                    wbuf[r, sl] = xv
                    if not final:
                        ybuf[r, sl] = xv * dv

        pltpu.sync_copy(wbuf, x_out.at[pl.ds(gr, RCH)])
        if not final:
            pltpu.sync_copy(ybuf, y_out.at[pl.ds(gr, RCH)])


def _combine_body(x0, x1, x2, x3, out, b0, b1, b2, b3):
    c = lax.axis_index("c")
    s = lax.axis_index("s")
    r0 = s * TS + c * (TS // NC)

    @pl.loop(0, TS // NC, step=RCB)
    def _(g):
        pltpu.sync_copy(x0.at[pl.ds(r0 + g, RCB)], b0)
        pltpu.sync_copy(x1.at[pl.ds(r0 + g, RCB)], b1)
        pltpu.sync_copy(x2.at[pl.ds(r0 + g, RCB)], b2)
        pltpu.sync_copy(x3.at[pl.ds(r0 + g, RCB)], b3)

        @pl.loop(0, RCB)
        def _(r):
            for k in range(D // 16):
                sl = pl.ds(k * 16, 16)
                b0[r, sl] = (b0[r, sl] + b1[r, sl]
                             + b2[r, sl] + b3[r, sl]) * ALPHA

        pltpu.sync_copy(b0, out.at[pl.ds(r0 + g, RCB)])


def _dot_body(out_t, row2d, col2d, res2d, abuf, bbuf, ridx, cidx,
              rbuf, tbuf):
    c = lax.axis_index("c")
    s = lax.axis_index("s")
    wid = s * NC + c
    iota = lax.iota(_i32, 16)
    zeros16 = jnp.zeros((16,), _f32)

    @pl.loop(0, DOT_PT)
    def _(jj):
        chunk = wid * DOT_PT + jj
        pltpu.sync_copy(row2d.at[chunk], ridx)
        pltpu.sync_copy(col2d.at[chunk], cidx)
        pltpu.sync_copy(out_t.at[ridx], abuf)
        pltpu.sync_copy(out_t.at[cidx], bbuf)

        @pl.loop(0, CHD // 16)
        def _(g):
            for e in range(16):
                r = g * 16 + e
                p = (abuf[r, pl.ds(0, 16)] * bbuf[r, pl.ds(0, 16)]
                     + abuf[r, pl.ds(16, 16)] * bbuf[r, pl.ds(16, 16)])
                q = (abuf[r, pl.ds(32, 16)] * bbuf[r, pl.ds(32, 16)]
                     + abuf[r, pl.ds(48, 16)] * bbuf[r, pl.ds(48, 16)])
                tbuf[e, pl.ds(0, 16)] = p + q
            tot = plsc.load_gather(tbuf, [iota, jnp.full((16,), 0, _i32)])
            for l in range(1, 16):
                tot = tot + plsc.load_gather(
                    tbuf, [iota, jnp.full((16,), l, _i32)])
            rbuf[pl.ds(g * 16, 16)] = tot

        pltpu.sync_copy(rbuf, res2d.at[chunk])


def kernel(edge_index, embedding_weight):
    row = edge_index[0]
    col = edge_index[1]
    pad = PAD_E - E
    rowp = jnp.pad(row, (0, pad))
    colp = jnp.pad(col, (0, pad), constant_values=PAD_COL)
    row2dl = rowp.reshape(NCHUNK_L, CHL)
    col2dl = colp.reshape(NCHUNK_L, CHL)
    row2dd = rowp.reshape(NCHUNK_D, CHD)
    col2dd = colp.reshape(NCHUNK_D, CHD)
    x0 = jnp.pad(embedding_weight, ((0, NPAD - N), (0, 0)))

    nd = jax.ShapeDtypeStruct((NPAD, D), _f32)

    k1 = pl.kernel(
        _k1_body,
        out_type=[jax.ShapeDtypeStruct((NPAD,), _f32), nd],
        mesh=_mesh(),
        compiler_params=_CP,
        scratch_types=[
            pltpu.VMEM((NPAD,), _f32),
            pltpu.VMEM_SHARED((NS, NPAD), _f32),
            pltpu.VMEM((TS,), _f32),
            pltpu.VMEM((RCH, D), _f32),
            pltpu.VMEM((CHL,), _i32),
        ])
    dis, y0 = k1(col2dl, x0)

    layer_scratch = [
        pltpu.VMEM_SHARED((ACC_ROWS, D), _f32),
        pltpu.VMEM((CHL, D), _f32),
        pltpu.VMEM((CHL,), _i32),
        pltpu.VMEM((CHL // 128, 128), _i32),
        pltpu.VMEM((CHL,), _i32),
        pltpu.VMEM((RCH, D), _f32),
        pltpu.VMEM((RCH,), _f32),
        pltpu.SemaphoreType.DMA,
        pltpu.SemaphoreType.DMA,
    ]
    klayer = pl.kernel(
        functools.partial(_layer_body, False),
        out_type=[nd, nd],
        mesh=_mesh(),
        compiler_params=_CP,
        scratch_types=layer_scratch)
    x1, y1 = klayer(y0, dis, row2dl, col2dl)
    x2, y2 = klayer(y1, dis, row2dl, col2dl)

    kfinal = pl.kernel(
        functools.partial(_layer_body, True),
        out_type=[nd, nd],
        mesh=_mesh(),
        compiler_params=_CP,
        scratch_types=layer_scratch)
    x3, _y3 = kfinal(y2, dis, row2dl, col2dl)

    kcombine = pl.kernel(
        _combine_body,
        out_type=[nd],
        mesh=_mesh(),
        compiler_params=_CP,
        scratch_types=[
            pltpu.VMEM((RCB, D), _f32),
            pltpu.VMEM((RCB, D), _f32),
            pltpu.VMEM((RCB, D), _f32),
            pltpu.VMEM((RCB, D), _f32),
        ])
    (out_t,) = kcombine(x0, x1, x2, x3)

    kdot = pl.kernel(
        _dot_body,
        out_type=[jax.ShapeDtypeStruct((NCHUNK_D, CHD), _f32)],
        mesh=_mesh(),
        compiler_params=_CP,
        scratch_types=[
            pltpu.VMEM((CHD, D), _f32),
            pltpu.VMEM((CHD, D), _f32),
            pltpu.VMEM((CHD,), _i32),
            pltpu.VMEM((CHD,), _i32),
            pltpu.VMEM((CHD,), _f32),
            pltpu.VMEM((16, 17), _f32),
        ])
    (res2d,) = kdot(out_t, row2dd, col2dd)
    return res2d.reshape(-1)[:E]


# speedup vs baseline: 9.0154x; 1.0744x over previous
import functools

import jax
import jax.numpy as jnp
from jax import lax
from jax.experimental import pallas as pl
from jax.experimental.pallas import tpu as pltpu
from jax.experimental.pallas import tpu_sc as plsc

N = 50000
D = 64
E = 800000
NUM_LAYERS = 3
ALPHA = 1.0 / (NUM_LAYERS + 1)

NC = 2
NS = 16
NPAD = 51200
TS = NPAD // NS
HALF = NPAD // NC
DUMP = HALF
ACC_ROWS = 25920
CHL = 256
CHD = 512
RCH = 80
RCB = 320
PAD_E = 802816
NCHUNK_L = PAD_E // CHL
NCHUNK_D = PAD_E // CHD
PROP_PT = NCHUNK_L // NS
DOT_PT = NCHUNK_D // (NC * NS)
PAD_COL = NPAD - 1

_i32 = jnp.int32
_f32 = jnp.float32

_CP = pltpu.CompilerParams(needs_layout_passes=False,
                           use_tc_tiling_on_sc=False)


def _mesh():
    return plsc.VectorSubcoreMesh(
        core_axis_name="c", subcore_axis_name="s",
        num_cores=NC, num_subcores=NS)


def _rsqrt16(x):
    i = plsc.bitcast(x, _i32)
    i = jnp.int32(0x5F3759DF) - lax.shift_right_logical(i, 1)
    y = plsc.bitcast(i, _f32)
    for _ in range(3):
        y = y * (1.5 - 0.5 * x * y * y)
    return y


def _k1_body(col2d, x0, dis_out, y0_out, hist, stage, wrow, xbuf, cbuf):
    c = lax.axis_index("c")
    s = lax.axis_index("s")
    zeros16 = jnp.zeros((16,), _f32)
    ones16 = jnp.ones((16,), _f32)
    iota = lax.iota(_i32, 16)

    @pl.loop(0, NPAD, step=16)
    def _(j):
        hist[pl.ds(j, 16)] = zeros16

    @pl.loop(0, PROP_PT)
    def _(jj):
        chunk = s * PROP_PT + jj
        pltpu.sync_copy(col2d.at[chunk], cbuf)
        for g in range(CHL // 16):
            cc = cbuf[pl.ds(g * 16, 16)]
            plsc.addupdate_scatter(hist, [cc], ones16)

    pltpu.sync_copy(hist, stage.at[s])
    plsc.subcore_barrier()

    base = s * TS
    @pl.loop(0, TS, step=16)
    def _(j):
        hist[pl.ds(j, 16)] = zeros16

    for w in range(NS):
        pltpu.sync_copy(stage.at[w, pl.ds(base, TS)], wrow)

        @pl.loop(0, TS, step=16)
        def _(j):
            hist[pl.ds(j, 16)] = hist[pl.ds(j, 16)] + wrow[pl.ds(j, 16)]

    @pl.loop(0, TS, step=16)
    def _(j):
        dv = hist[pl.ds(j, 16)]
        r = _rsqrt16(jnp.maximum(dv, 1.0))
        hist[pl.ds(j, 16)] = jnp.where(dv > 0, r, 0.0)

    @pl.when(c == 0)
    def _():
        pltpu.sync_copy(hist.at[pl.ds(0, TS)], dis_out.at[pl.ds(base, TS)])

    r0 = s * TS + c * (TS // NC)

    @pl.loop(0, TS // NC, step=RCH)
    def _(g):
        pltpu.sync_copy(x0.at[pl.ds(r0 + g, RCH)], xbuf)

        @pl.loop(0, RCH, step=16)
        def _(rr):
            dvec = hist[pl.ds(c * (TS // NC) + g + rr, 16)]
            for e in range(16):
                dv = dvec[jnp.full((16,), e, _i32)]
                r = rr + e
                for k in range(D // 16):
                    sl = pl.ds(k * 16, 16)
                    xbuf[r, sl] = xbuf[r, sl] * dv

        pltpu.sync_copy(xbuf, y0_out.at[pl.ds(r0 + g, RCH)])


def _layer_body(final, y_prev, dis, row2d, col2d, x_out, y_out,
                acc, gbuf, ridx, lidx, cbuf, ybuf, dbuf, semg, sems):
    c = lax.axis_index("c")
    s = lax.axis_index("s")
    base = c * HALF
    zeros16 = jnp.zeros((16,), _f32)
    iota = lax.iota(_i32, 16)

    @pl.loop(0, 162)
    def _(r):
        for k in range(D // 16):
            gbuf[r, pl.ds(k * 16, 16)] = zeros16

    @pl.loop(0, ACC_ROWS // NS, step=162)
    def _(j):
        pltpu.sync_copy(gbuf.at[pl.ds(0, 162)],
                        acc.at[pl.ds(s * (ACC_ROWS // NS) + j, 162)])

    plsc.subcore_barrier()

    @pl.loop(0, PROP_PT)
    def _(jj):
        chunk = s * PROP_PT + jj
        pltpu.sync_copy(row2d.at[chunk], ridx)
        pltpu.sync_copy(col2d.at[chunk], cbuf)
        neg1 = jnp.full((16,), -1, _i32)
        for g in range(CHL // 16):
            cc = cbuf[pl.ds(g * 16, 16)]
            rv = ridx[pl.ds(g * 16, 16)]
            ok = (cc >= base) & (cc < base + HALF)
            ridx[pl.ds(g * 16, 16)] = jnp.where(ok, rv, neg1)
            lidx[g // 8, pl.ds((g % 8) * 16, 16)] = jnp.where(
                ok, cc - base, neg1)

        @pl.loop(0, CHL // 128)
        def _(k):
            pltpu.async_copy(
                y_prev.at[plsc.Indices(ridx.at[pl.ds(k * 128, 128)],
                                       ignored_value=-1)],
                gbuf.at[pl.ds(k * 128, 128)], semg)

        @pl.loop(0, CHL // 128)
        def _(k):
            pltpu.make_async_copy(
                y_prev.at[plsc.Indices(ridx.at[pl.ds(k * 128, 128)],
                                       ignored_value=-1)],
                gbuf.at[pl.ds(k * 128, 128)], semg).wait()

        @pl.loop(0, CHL // 128)
        def _(k):
            pltpu.async_copy(gbuf.at[pl.ds(k * 128, 128)],
                             acc.at[plsc.Indices(lidx.at[k],
                                                 ignored_value=-1)],
                             sems, add=True)

        @pl.loop(0, CHL // 128)
        def _(k):
            pltpu.make_async_copy(gbuf.at[pl.ds(k * 128, 128)],
                                  acc.at[plsc.Indices(lidx.at[k],
                                                      ignored_value=-1)],
                                  sems).wait()

    plsc.subcore_barrier()

    r0l = s * (HALF // NS)

    @pl.loop(0, HALF // NS, step=RCH)
    def _(g):
        lr = r0l + g
        gr = base + lr
        wbuf = gbuf.at[pl.ds(0, RCH)]
        pltpu.sync_copy(acc.at[pl.ds(lr, RCH)], wbuf)
        pltpu.sync_copy(dis.at[pl.ds(gr, RCH)], dbuf)

        @pl.loop(0, RCH, step=16)
        def _(rr):
            dvec = dbuf[pl.ds(rr, 16)]
            for e in range(16):
                dv = dvec[jnp.full((16,), e, _i32)]
                r = rr + e
                for k in range(D // 16):
                    sl = pl.ds(k * 16, 16)
                    xv = wbuf[r, sl] * dv
                    wbuf[r, sl] = xv
                    if not final:
                        ybuf[r, sl] = xv * dv

        pltpu.sync_copy(wbuf, x_out.at[pl.ds(gr, RCH)])
        if not final:
            pltpu.sync_copy(ybuf, y_out.at[pl.ds(gr, RCH)])


def _combine_body(x0, x1, x2, x3, out, b0, b1, b2, b3):
    c = lax.axis_index("c")
    s = lax.axis_index("s")
    r0 = s * TS + c * (TS // NC)

    @pl.loop(0, TS // NC, step=RCB)
    def _(g):
        pltpu.sync_copy(x0.at[pl.ds(r0 + g, RCB)], b0)
        pltpu.sync_copy(x1.at[pl.ds(r0 + g, RCB)], b1)
        pltpu.sync_copy(x2.at[pl.ds(r0 + g, RCB)], b2)
        pltpu.sync_copy(x3.at[pl.ds(r0 + g, RCB)], b3)

        @pl.loop(0, RCB)
        def _(r):
            for k in range(D // 16):
                sl = pl.ds(k * 16, 16)
                b0[r, sl] = (b0[r, sl] + b1[r, sl]
                             + b2[r, sl] + b3[r, sl]) * ALPHA

        pltpu.sync_copy(b0, out.at[pl.ds(r0 + g, RCB)])


def _dot_body(out_t, row2d, col2d, res2d, abuf, bbuf, ridx, cidx,
              rbuf, tbuf):
    c = lax.axis_index("c")
    s = lax.axis_index("s")
    wid = s * NC + c
    iota = lax.iota(_i32, 16)
    zeros16 = jnp.zeros((16,), _f32)

    @pl.loop(0, DOT_PT)
    def _(jj):
        chunk = wid * DOT_PT + jj
        pltpu.sync_copy(row2d.at[chunk], ridx)
        pltpu.sync_copy(col2d.at[chunk], cidx)
        pltpu.sync_copy(out_t.at[ridx], abuf)
        pltpu.sync_copy(out_t.at[cidx], bbuf)

        @pl.loop(0, CHD // 16)
        def _(g):
            for e in range(16):
                r = g * 16 + e
                p = (abuf[r, pl.ds(0, 16)] * bbuf[r, pl.ds(0, 16)]
                     + abuf[r, pl.ds(16, 16)] * bbuf[r, pl.ds(16, 16)])
                q = (abuf[r, pl.ds(32, 16)] * bbuf[r, pl.ds(32, 16)]
                     + abuf[r, pl.ds(48, 16)] * bbuf[r, pl.ds(48, 16)])
                tbuf[e, pl.ds(0, 16)] = p + q
            tot = plsc.load_gather(tbuf, [iota, jnp.full((16,), 0, _i32)])
            for l in range(1, 16):
                tot = tot + plsc.load_gather(
                    tbuf, [iota, jnp.full((16,), l, _i32)])
            rbuf[pl.ds(g * 16, 16)] = tot

        pltpu.sync_copy(rbuf, res2d.at[chunk])


def kernel(edge_index, embedding_weight):
    row = edge_index[0]
    col = edge_index[1]
    pad = PAD_E - E
    rowp = jnp.pad(row, (0, pad))
    colp = jnp.pad(col, (0, pad), constant_values=PAD_COL)
    row2dl = rowp.reshape(NCHUNK_L, CHL)
    col2dl = colp.reshape(NCHUNK_L, CHL)
    row2dd = rowp.reshape(NCHUNK_D, CHD)
    col2dd = colp.reshape(NCHUNK_D, CHD)
    x0 = jnp.pad(embedding_weight, ((0, NPAD - N), (0, 0)))

    nd = jax.ShapeDtypeStruct((NPAD, D), _f32)

    k1 = pl.kernel(
        _k1_body,
        out_type=[jax.ShapeDtypeStruct((NPAD,), _f32), nd],
        mesh=_mesh(),
        compiler_params=_CP,
        scratch_types=[
            pltpu.VMEM((NPAD,), _f32),
            pltpu.VMEM_SHARED((NS, NPAD), _f32),
            pltpu.VMEM((TS,), _f32),
            pltpu.VMEM((RCH, D), _f32),
            pltpu.VMEM((CHL,), _i32),
        ])
    dis, y0 = k1(col2dl, x0)

    layer_scratch = [
        pltpu.VMEM_SHARED((ACC_ROWS, D), _f32),
        pltpu.VMEM((CHL, D), _f32),
        pltpu.VMEM((CHL,), _i32),
        pltpu.VMEM((CHL // 128, 128), _i32),
        pltpu.VMEM((CHL,), _i32),
        pltpu.VMEM((RCH, D), _f32),
        pltpu.VMEM((RCH,), _f32),
        pltpu.SemaphoreType.DMA,
        pltpu.SemaphoreType.DMA,
    ]
    klayer = pl.kernel(
        functools.partial(_layer_body, False),
        out_type=[nd, nd],
        mesh=_mesh(),
        compiler_params=_CP,
        scratch_types=layer_scratch)
    x1, y1 = klayer(y0, dis, row2dl, col2dl)
    x2, y2 = klayer(y1, dis, row2dl, col2dl)

    kfinal = pl.kernel(
        functools.partial(_layer_body, True),
        out_type=[nd, nd],
        mesh=_mesh(),
        compiler_params=_CP,
        scratch_types=layer_scratch)
    x3, _y3 = kfinal(y2, dis, row2dl, col2dl)

    kcombine = pl.kernel(
        _combine_body,
        out_type=[nd],
        mesh=_mesh(),
        compiler_params=_CP,
        scratch_types=[
            pltpu.VMEM((RCB, D), _f32),
            pltpu.VMEM((RCB, D), _f32),
            pltpu.VMEM((RCB, D), _f32),
            pltpu.VMEM((RCB, D), _f32),
        ])
    (out_t,) = kcombine(x0, x1, x2, x3)

    kdot = pl.kernel(
        _dot_body,
        out_type=[jax.ShapeDtypeStruct((NCHUNK_D, CHD), _f32)],
        mesh=_mesh(),
        compiler_params=_CP,
        scratch_types=[
            pltpu.VMEM((CHD, D), _f32),
            pltpu.VMEM((CHD, D), _f32),
            pltpu.VMEM((CHD,), _i32),
            pltpu.VMEM((CHD,), _i32),
            pltpu.VMEM((CHD,), _f32),
            pltpu.VMEM((16, 17), _f32),
        ])
    (res2d,) = kdot(out_t, row2dd, col2dd)
    return res2d.reshape(-1)[:E]


# speedup vs baseline: 10.7602x; 1.1935x over previous
import functools

import jax
import jax.numpy as jnp
from jax import lax
from jax.experimental import pallas as pl
from jax.experimental.pallas import tpu as pltpu
from jax.experimental.pallas import tpu_sc as plsc

N = 50000
D = 64
E = 800000
NUM_LAYERS = 3
ALPHA = 1.0 / (NUM_LAYERS + 1)

NC = 2
NS = 16
NPAD = 51200
TS = NPAD // NS
HALF = NPAD // NC
DUMP = HALF
ACC_ROWS = 25920
CHL = 128
CHD = 512
RCH = 80
RCB = 320
PAD_E = 802816
NCHUNK_L = PAD_E // CHL
NCHUNK_D = PAD_E // CHD
PROP_PT = NCHUNK_L // NS
DOT_PT = NCHUNK_D // (NC * NS)
PAD_COL = NPAD - 1

_i32 = jnp.int32
_f32 = jnp.float32

_CP = pltpu.CompilerParams(needs_layout_passes=False,
                           use_tc_tiling_on_sc=False)


def _mesh():
    return plsc.VectorSubcoreMesh(
        core_axis_name="c", subcore_axis_name="s",
        num_cores=NC, num_subcores=NS)


def _rsqrt16(x):
    i = plsc.bitcast(x, _i32)
    i = jnp.int32(0x5F3759DF) - lax.shift_right_logical(i, 1)
    y = plsc.bitcast(i, _f32)
    for _ in range(3):
        y = y * (1.5 - 0.5 * x * y * y)
    return y


def _k1_body(col2d, x0, dis_out, y0_out, hist, stage, wrow, xbuf, cbuf):
    c = lax.axis_index("c")
    s = lax.axis_index("s")
    zeros16 = jnp.zeros((16,), _f32)
    ones16 = jnp.ones((16,), _f32)
    iota = lax.iota(_i32, 16)

    @pl.loop(0, NPAD, step=16)
    def _(j):
        hist[pl.ds(j, 16)] = zeros16

    @pl.loop(0, PROP_PT)
    def _(jj):
        chunk = s * PROP_PT + jj
        pltpu.sync_copy(col2d.at[chunk], cbuf)
        for g in range(CHL // 16):
            cc = cbuf[pl.ds(g * 16, 16)]
            plsc.addupdate_scatter(hist, [cc], ones16)

    pltpu.sync_copy(hist, stage.at[s])
    plsc.subcore_barrier()

    base = s * TS
    @pl.loop(0, TS, step=16)
    def _(j):
        hist[pl.ds(j, 16)] = zeros16

    for w in range(NS):
        pltpu.sync_copy(stage.at[w, pl.ds(base, TS)], wrow)

        @pl.loop(0, TS, step=16)
        def _(j):
            hist[pl.ds(j, 16)] = hist[pl.ds(j, 16)] + wrow[pl.ds(j, 16)]

    @pl.loop(0, TS, step=16)
    def _(j):
        dv = hist[pl.ds(j, 16)]
        r = _rsqrt16(jnp.maximum(dv, 1.0))
        hist[pl.ds(j, 16)] = jnp.where(dv > 0, r, 0.0)

    @pl.when(c == 0)
    def _():
        pltpu.sync_copy(hist.at[pl.ds(0, TS)], dis_out.at[pl.ds(base, TS)])

    r0 = s * TS + c * (TS // NC)

    @pl.loop(0, TS // NC, step=RCH)
    def _(g):
        pltpu.sync_copy(x0.at[pl.ds(r0 + g, RCH)], xbuf)

        @pl.loop(0, RCH, step=16)
        def _(rr):
            dvec = hist[pl.ds(c * (TS // NC) + g + rr, 16)]
            for e in range(16):
                dv = dvec[jnp.full((16,), e, _i32)]
                r = rr + e
                for k in range(D // 16):
                    sl = pl.ds(k * 16, 16)
                    xbuf[r, sl] = xbuf[r, sl] * dv

        pltpu.sync_copy(xbuf, y0_out.at[pl.ds(r0 + g, RCH)])


def _layer_body(final, y_prev, dis, row2d, col2d, x_out, y_out,
                acc, gbuf0, gbuf1, ridx0, ridx1, cbuf0, cbuf1,
                lidx0, lidx1, ybuf, dbuf,
                semi0, semi1, semg0, semg1, sems0, sems1):
    c = lax.axis_index("c")
    s = lax.axis_index("s")
    base = c * HALF
    zeros16 = jnp.zeros((16,), _f32)
    neg1 = jnp.full((16,), -1, _i32)
    gbuf = (gbuf0, gbuf1)
    ridx = (ridx0, ridx1)
    cbuf = (cbuf0, cbuf1)
    lidx = (lidx0, lidx1)
    semi = (semi0, semi1)
    semg = (semg0, semg1)
    sems = (sems0, sems1)

    @pl.loop(0, 108)
    def _(r):
        for k in range(D // 16):
            gbuf0[r, pl.ds(k * 16, 16)] = zeros16

    zsrc = gbuf0.at[pl.ds(0, 108)]

    @pl.loop(0, ACC_ROWS // NS, step=108)
    def _(j):
        pltpu.async_copy(
            zsrc, acc.at[pl.ds(s * (ACC_ROWS // NS) + j, 108)], semi0)

    @pl.loop(0, ACC_ROWS // NS, step=108)
    def _(j):
        pltpu.make_async_copy(
            zsrc, acc.at[pl.ds(s * (ACC_ROWS // NS) + j, 108)],
            semi0).wait()

    plsc.subcore_barrier()

    c0 = s * PROP_PT

    def fire_idx(ch, b):
        pltpu.async_copy(row2d.at[ch], ridx[b], semi[b])
        pltpu.async_copy(col2d.at[ch], cbuf[b], semi[b])

    def wait_idx(ch, b):
        pltpu.make_async_copy(row2d.at[ch], ridx[b], semi[b]).wait()
        pltpu.make_async_copy(col2d.at[ch], cbuf[b], semi[b]).wait()

    def fire_gather(b):
        pltpu.async_copy(
            y_prev.at[plsc.Indices(ridx[b], ignored_value=-1)],
            gbuf[b], semg[b])

    def wait_gather(b):
        pltpu.make_async_copy(
            y_prev.at[plsc.Indices(ridx[b], ignored_value=-1)],
            gbuf[b], semg[b]).wait()

    def fire_scatter(b):
        pltpu.async_copy(
            gbuf[b], acc.at[plsc.Indices(lidx[b].at[0], ignored_value=-1)],
            sems[b], add=True)

    def wait_scatter(b):
        pltpu.make_async_copy(
            gbuf[b], acc.at[plsc.Indices(lidx[b].at[0], ignored_value=-1)],
            sems[b]).wait()

    def compute_masks(b):
        for g in range(CHL // 16):
            cc = cbuf[b][pl.ds(g * 16, 16)]
            rv = ridx[b][pl.ds(g * 16, 16)]
            ok = (cc >= base) & (cc < base + HALF)
            ridx[b][pl.ds(g * 16, 16)] = jnp.where(ok, rv, neg1)
            lidx[b][0, pl.ds(g * 16, 16)] = jnp.where(ok, cc - base, neg1)

    fire_idx(c0, 0)
    fire_idx(c0 + 1, 1)

    @pl.loop(0, PROP_PT // 2)
    def _(k):
        for b in (0, 1):
            j = 2 * k + b

            if b == 0:
                @pl.when(k >= 1)
                def _():
                    wait_gather(1)
                    fire_scatter(1)
                    fire_idx(c0 + j + 1, 1)
            else:
                wait_gather(0)
                fire_scatter(0)

                @pl.when(k < PROP_PT // 2 - 1)
                def _():
                    fire_idx(c0 + j + 1, 0)

            wait_idx(c0 + j, b)

            @pl.when(k >= 1)
            def _():
                wait_scatter(b)

            compute_masks(b)
            fire_gather(b)

    wait_gather(1)
    fire_scatter(1)
    wait_scatter(0)
    wait_scatter(1)

    plsc.subcore_barrier()

    r0l = s * (HALF // NS)

    @pl.loop(0, HALF // NS, step=RCH)
    def _(g):
        lr = r0l + g
        gr = base + lr
        wbuf = gbuf0.at[pl.ds(0, RCH)]
        pltpu.sync_copy(acc.at[pl.ds(lr, RCH)], wbuf)
        pltpu.sync_copy(dis.at[pl.ds(gr, RCH)], dbuf)

        @pl.loop(0, RCH, step=16)
        def _(rr):
            dvec = dbuf[pl.ds(rr, 16)]
            for e in range(16):
                dv = dvec[jnp.full((16,), e, _i32)]
                r = rr + e
                for k in range(D // 16):
                    sl = pl.ds(k * 16, 16)
                    xv = wbuf[r, sl] * dv
                    wbuf[r, sl] = xv
                    if not final:
                        ybuf[r, sl] = xv * dv

        pltpu.sync_copy(wbuf, x_out.at[pl.ds(gr, RCH)])
        if not final:
            pltpu.sync_copy(ybuf, y_out.at[pl.ds(gr, RCH)])


def _combine_body(x0, x1, x2, x3, out, b0, b1, b2, b3):
    c = lax.axis_index("c")
    s = lax.axis_index("s")
    r0 = s * TS + c * (TS // NC)

    @pl.loop(0, TS // NC, step=RCB)
    def _(g):
        pltpu.sync_copy(x0.at[pl.ds(r0 + g, RCB)], b0)
        pltpu.sync_copy(x1.at[pl.ds(r0 + g, RCB)], b1)
        pltpu.sync_copy(x2.at[pl.ds(r0 + g, RCB)], b2)
        pltpu.sync_copy(x3.at[pl.ds(r0 + g, RCB)], b3)

        @pl.loop(0, RCB)
        def _(r):
            for k in range(D // 16):
                sl = pl.ds(k * 16, 16)
                b0[r, sl] = (b0[r, sl] + b1[r, sl]
                             + b2[r, sl] + b3[r, sl]) * ALPHA

        pltpu.sync_copy(b0, out.at[pl.ds(r0 + g, RCB)])


def _dot_body(out_t, row2d, col2d, res2d, abuf, bbuf, ridx, cidx,
              rbuf, tbuf):
    c = lax.axis_index("c")
    s = lax.axis_index("s")
    wid = s * NC + c
    iota = lax.iota(_i32, 16)
    zeros16 = jnp.zeros((16,), _f32)

    @pl.loop(0, DOT_PT)
    def _(jj):
        chunk = wid * DOT_PT + jj
        pltpu.sync_copy(row2d.at[chunk], ridx)
        pltpu.sync_copy(col2d.at[chunk], cidx)
        pltpu.sync_copy(out_t.at[ridx], abuf)
        pltpu.sync_copy(out_t.at[cidx], bbuf)

        @pl.loop(0, CHD // 16)
        def _(g):
            for e in range(16):
                r = g * 16 + e
                p = (abuf[r, pl.ds(0, 16)] * bbuf[r, pl.ds(0, 16)]
                     + abuf[r, pl.ds(16, 16)] * bbuf[r, pl.ds(16, 16)])
                q = (abuf[r, pl.ds(32, 16)] * bbuf[r, pl.ds(32, 16)]
                     + abuf[r, pl.ds(48, 16)] * bbuf[r, pl.ds(48, 16)])
                tbuf[e, pl.ds(0, 16)] = p + q
            tot = plsc.load_gather(tbuf, [iota, jnp.full((16,), 0, _i32)])
            for l in range(1, 16):
                tot = tot + plsc.load_gather(
                    tbuf, [iota, jnp.full((16,), l, _i32)])
            rbuf[pl.ds(g * 16, 16)] = tot

        pltpu.sync_copy(rbuf, res2d.at[chunk])


def kernel(edge_index, embedding_weight):
    row = edge_index[0]
    col = edge_index[1]
    pad = PAD_E - E
    rowp = jnp.pad(row, (0, pad))
    colp = jnp.pad(col, (0, pad), constant_values=PAD_COL)
    row2dl = rowp.reshape(NCHUNK_L, CHL)
    col2dl = colp.reshape(NCHUNK_L, CHL)
    row2dd = rowp.reshape(NCHUNK_D, CHD)
    col2dd = colp.reshape(NCHUNK_D, CHD)
    x0 = jnp.pad(embedding_weight, ((0, NPAD - N), (0, 0)))

    nd = jax.ShapeDtypeStruct((NPAD, D), _f32)

    k1 = pl.kernel(
        _k1_body,
        out_type=[jax.ShapeDtypeStruct((NPAD,), _f32), nd],
        mesh=_mesh(),
        compiler_params=_CP,
        scratch_types=[
            pltpu.VMEM((NPAD,), _f32),
            pltpu.VMEM_SHARED((NS, NPAD), _f32),
            pltpu.VMEM((TS,), _f32),
            pltpu.VMEM((RCH, D), _f32),
            pltpu.VMEM((CHL,), _i32),
        ])
    dis, y0 = k1(col2dl, x0)

    layer_scratch = [
        pltpu.VMEM_SHARED((ACC_ROWS, D), _f32),
        pltpu.VMEM((CHL, D), _f32),
        pltpu.VMEM((CHL, D), _f32),
        pltpu.VMEM((CHL,), _i32),
        pltpu.VMEM((CHL,), _i32),
        pltpu.VMEM((CHL,), _i32),
        pltpu.VMEM((CHL,), _i32),
        pltpu.VMEM((1, CHL), _i32),
        pltpu.VMEM((1, CHL), _i32),
        pltpu.VMEM((RCH, D), _f32),
        pltpu.VMEM((RCH,), _f32),
        pltpu.SemaphoreType.DMA,
        pltpu.SemaphoreType.DMA,
        pltpu.SemaphoreType.DMA,
        pltpu.SemaphoreType.DMA,
        pltpu.SemaphoreType.DMA,
        pltpu.SemaphoreType.DMA,
    ]
    klayer = pl.kernel(
        functools.partial(_layer_body, False),
        out_type=[nd, nd],
        mesh=_mesh(),
        compiler_params=_CP,
        scratch_types=layer_scratch)
    x1, y1 = klayer(y0, dis, row2dl, col2dl)
    x2, y2 = klayer(y1, dis, row2dl, col2dl)

    kfinal = pl.kernel(
        functools.partial(_layer_body, True),
        out_type=[nd, nd],
        mesh=_mesh(),
        compiler_params=_CP,
        scratch_types=layer_scratch)
    x3, _y3 = kfinal(y2, dis, row2dl, col2dl)

    kcombine = pl.kernel(
        _combine_body,
        out_type=[nd],
        mesh=_mesh(),
        compiler_params=_CP,
        scratch_types=[
            pltpu.VMEM((RCB, D), _f32),
            pltpu.VMEM((RCB, D), _f32),
            pltpu.VMEM((RCB, D), _f32),
            pltpu.VMEM((RCB, D), _f32),
        ])
    (out_t,) = kcombine(x0, x1, x2, x3)

    kdot = pl.kernel(
        _dot_body,
        out_type=[jax.ShapeDtypeStruct((NCHUNK_D, CHD), _f32)],
        mesh=_mesh(),
        compiler_params=_CP,
        scratch_types=[
            pltpu.VMEM((CHD, D), _f32),
            pltpu.VMEM((CHD, D), _f32),
            pltpu.VMEM((CHD,), _i32),
            pltpu.VMEM((CHD,), _i32),
            pltpu.VMEM((CHD,), _f32),
            pltpu.VMEM((16, 17), _f32),
        ])
    (res2d,) = kdot(out_t, row2dd, col2dd)
    return res2d.reshape(-1)[:E]


# speedup vs baseline: 11.9589x; 1.1114x over previous
import functools

import jax
import jax.numpy as jnp
from jax import lax
from jax.experimental import pallas as pl
from jax.experimental.pallas import tpu as pltpu
from jax.experimental.pallas import tpu_sc as plsc

N = 50000
D = 64
E = 800000
NUM_LAYERS = 3
ALPHA = 1.0 / (NUM_LAYERS + 1)

NC = 2
NS = 16
NPAD = 51200
TS = NPAD // NS
HALF = NPAD // NC
DUMP = HALF
ACC_ROWS = 25920
CHL = 128
CHD = 256
RCH = 80
RCB = 320
PAD_E = 802816
NCHUNK_L = PAD_E // CHL
NCHUNK_D = PAD_E // CHD
PROP_PT = NCHUNK_L // NS
DOT_PT = NCHUNK_D // (NC * NS)
PAD_COL = NPAD - 1

_i32 = jnp.int32
_f32 = jnp.float32

_CP = pltpu.CompilerParams(needs_layout_passes=False,
                           use_tc_tiling_on_sc=False)


def _mesh():
    return plsc.VectorSubcoreMesh(
        core_axis_name="c", subcore_axis_name="s",
        num_cores=NC, num_subcores=NS)


def _rsqrt16(x):
    i = plsc.bitcast(x, _i32)
    i = jnp.int32(0x5F3759DF) - lax.shift_right_logical(i, 1)
    y = plsc.bitcast(i, _f32)
    for _ in range(3):
        y = y * (1.5 - 0.5 * x * y * y)
    return y


def _k1_body(col2d, x0, dis_out, y0_out, hist, stage, wrow, xbuf, cbuf):
    c = lax.axis_index("c")
    s = lax.axis_index("s")
    zeros16 = jnp.zeros((16,), _f32)
    ones16 = jnp.ones((16,), _f32)
    iota = lax.iota(_i32, 16)

    @pl.loop(0, NPAD, step=16)
    def _(j):
        hist[pl.ds(j, 16)] = zeros16

    @pl.loop(0, PROP_PT)
    def _(jj):
        chunk = s * PROP_PT + jj
        pltpu.sync_copy(col2d.at[chunk], cbuf)
        for g in range(CHL // 16):
            cc = cbuf[pl.ds(g * 16, 16)]
            plsc.addupdate_scatter(hist, [cc], ones16)

    pltpu.sync_copy(hist, stage.at[s])
    plsc.subcore_barrier()

    base = s * TS
    @pl.loop(0, TS, step=16)
    def _(j):
        hist[pl.ds(j, 16)] = zeros16

    for w in range(NS):
        pltpu.sync_copy(stage.at[w, pl.ds(base, TS)], wrow)

        @pl.loop(0, TS, step=16)
        def _(j):
            hist[pl.ds(j, 16)] = hist[pl.ds(j, 16)] + wrow[pl.ds(j, 16)]

    @pl.loop(0, TS, step=16)
    def _(j):
        dv = hist[pl.ds(j, 16)]
        r = _rsqrt16(jnp.maximum(dv, 1.0))
        hist[pl.ds(j, 16)] = jnp.where(dv > 0, r, 0.0)

    @pl.when(c == 0)
    def _():
        pltpu.sync_copy(hist.at[pl.ds(0, TS)], dis_out.at[pl.ds(base, TS)])

    r0 = s * TS + c * (TS // NC)

    @pl.loop(0, TS // NC, step=RCH)
    def _(g):
        pltpu.sync_copy(x0.at[pl.ds(r0 + g, RCH)], xbuf)

        @pl.loop(0, RCH, step=16)
        def _(rr):
            dvec = hist[pl.ds(c * (TS // NC) + g + rr, 16)]
            for e in range(16):
                dv = dvec[jnp.full((16,), e, _i32)]
                r = rr + e
                for k in range(D // 16):
                    sl = pl.ds(k * 16, 16)
                    xbuf[r, sl] = xbuf[r, sl] * dv

        pltpu.sync_copy(xbuf, y0_out.at[pl.ds(r0 + g, RCH)])


def _layer_body(final, y_prev, dis, row2d, col2d, x_out, y_out,
                acc, gbuf0, gbuf1, ridx0, ridx1, cbuf0, cbuf1,
                lidx0, lidx1, ybuf, dbuf,
                semi0, semi1, semg0, semg1, sems0, sems1):
    c = lax.axis_index("c")
    s = lax.axis_index("s")
    base = c * HALF
    zeros16 = jnp.zeros((16,), _f32)
    neg1 = jnp.full((16,), -1, _i32)
    gbuf = (gbuf0, gbuf1)
    ridx = (ridx0, ridx1)
    cbuf = (cbuf0, cbuf1)
    lidx = (lidx0, lidx1)
    semi = (semi0, semi1)
    semg = (semg0, semg1)
    sems = (sems0, sems1)

    @pl.loop(0, 108)
    def _(r):
        for k in range(D // 16):
            gbuf0[r, pl.ds(k * 16, 16)] = zeros16

    zsrc = gbuf0.at[pl.ds(0, 108)]

    @pl.loop(0, ACC_ROWS // NS, step=108)
    def _(j):
        pltpu.async_copy(
            zsrc, acc.at[pl.ds(s * (ACC_ROWS // NS) + j, 108)], semi0)

    @pl.loop(0, ACC_ROWS // NS, step=108)
    def _(j):
        pltpu.make_async_copy(
            zsrc, acc.at[pl.ds(s * (ACC_ROWS // NS) + j, 108)],
            semi0).wait()

    plsc.subcore_barrier()

    c0 = s * PROP_PT

    def fire_idx(ch, b):
        pltpu.async_copy(row2d.at[ch], ridx[b], semi[b])
        pltpu.async_copy(col2d.at[ch], cbuf[b], semi[b])

    def wait_idx(ch, b):
        pltpu.make_async_copy(row2d.at[ch], ridx[b], semi[b]).wait()
        pltpu.make_async_copy(col2d.at[ch], cbuf[b], semi[b]).wait()

    def fire_gather(b):
        pltpu.async_copy(
            y_prev.at[plsc.Indices(ridx[b], ignored_value=-1)],
            gbuf[b], semg[b])

    def wait_gather(b):
        pltpu.make_async_copy(
            y_prev.at[plsc.Indices(ridx[b], ignored_value=-1)],
            gbuf[b], semg[b]).wait()

    def fire_scatter(b):
        pltpu.async_copy(
            gbuf[b], acc.at[plsc.Indices(lidx[b].at[0], ignored_value=-1)],
            sems[b], add=True)

    def wait_scatter(b):
        pltpu.make_async_copy(
            gbuf[b], acc.at[plsc.Indices(lidx[b].at[0], ignored_value=-1)],
            sems[b]).wait()

    def compute_masks(b):
        for g in range(CHL // 16):
            cc = cbuf[b][pl.ds(g * 16, 16)]
            rv = ridx[b][pl.ds(g * 16, 16)]
            ok = (cc >= base) & (cc < base + HALF)
            ridx[b][pl.ds(g * 16, 16)] = jnp.where(ok, rv, neg1)
            lidx[b][0, pl.ds(g * 16, 16)] = jnp.where(ok, cc - base, neg1)

    fire_idx(c0, 0)
    fire_idx(c0 + 1, 1)

    @pl.loop(0, PROP_PT // 2)
    def _(k):
        for b in (0, 1):
            j = 2 * k + b

            if b == 0:
                @pl.when(k >= 1)
                def _():
                    wait_gather(1)
                    fire_scatter(1)
                    fire_idx(c0 + j + 1, 1)
            else:
                wait_gather(0)
                fire_scatter(0)

                @pl.when(k < PROP_PT // 2 - 1)
                def _():
                    fire_idx(c0 + j + 1, 0)

            wait_idx(c0 + j, b)

            @pl.when(k >= 1)
            def _():
                wait_scatter(b)

            compute_masks(b)
            fire_gather(b)

    wait_gather(1)
    fire_scatter(1)
    wait_scatter(0)
    wait_scatter(1)

    plsc.subcore_barrier()

    r0l = s * (HALF // NS)

    @pl.loop(0, HALF // NS, step=RCH)
    def _(g):
        lr = r0l + g
        gr = base + lr
        wbuf = gbuf0.at[pl.ds(0, RCH)]
        pltpu.sync_copy(acc.at[pl.ds(lr, RCH)], wbuf)
        pltpu.sync_copy(dis.at[pl.ds(gr, RCH)], dbuf)

        @pl.loop(0, RCH, step=16)
        def _(rr):
            dvec = dbuf[pl.ds(rr, 16)]
            for e in range(16):
                dv = dvec[jnp.full((16,), e, _i32)]
                r = rr + e
                for k in range(D // 16):
                    sl = pl.ds(k * 16, 16)
                    xv = wbuf[r, sl] * dv
                    wbuf[r, sl] = xv
                    if not final:
                        ybuf[r, sl] = xv * dv

        pltpu.sync_copy(wbuf, x_out.at[pl.ds(gr, RCH)])
        if not final:
            pltpu.sync_copy(ybuf, y_out.at[pl.ds(gr, RCH)])


def _combine_body(x0, x1, x2, x3, out, b0, b1, b2, b3):
    c = lax.axis_index("c")
    s = lax.axis_index("s")
    r0 = s * TS + c * (TS // NC)

    @pl.loop(0, TS // NC, step=RCB)
    def _(g):
        pltpu.sync_copy(x0.at[pl.ds(r0 + g, RCB)], b0)
        pltpu.sync_copy(x1.at[pl.ds(r0 + g, RCB)], b1)
        pltpu.sync_copy(x2.at[pl.ds(r0 + g, RCB)], b2)
        pltpu.sync_copy(x3.at[pl.ds(r0 + g, RCB)], b3)

        @pl.loop(0, RCB)
        def _(r):
            for k in range(D // 16):
                sl = pl.ds(k * 16, 16)
                b0[r, sl] = (b0[r, sl] + b1[r, sl]
                             + b2[r, sl] + b3[r, sl]) * ALPHA

        pltpu.sync_copy(b0, out.at[pl.ds(r0 + g, RCB)])


def _dot_body(out_t, row2d, col2d, res2d, abuf0, abuf1, bbuf0, bbuf1,
              ridx0, ridx1, cidx0, cidx1, rbuf0, rbuf1, tbuf,
              semi0, semi1, semg0, semg1, semr0, semr1):
    c = lax.axis_index("c")
    s = lax.axis_index("s")
    wid = s * NC + c
    iota = lax.iota(_i32, 16)
    abuf = (abuf0, abuf1)
    bbuf = (bbuf0, bbuf1)
    ridx = (ridx0, ridx1)
    cidx = (cidx0, cidx1)
    rbuf = (rbuf0, rbuf1)
    semi = (semi0, semi1)
    semg = (semg0, semg1)
    semr = (semr0, semr1)
    c0 = wid * DOT_PT

    def fire_idx(ch, b):
        pltpu.async_copy(row2d.at[ch], ridx[b], semi[b])
        pltpu.async_copy(col2d.at[ch], cidx[b], semi[b])

    def wait_idx(ch, b):
        pltpu.make_async_copy(row2d.at[ch], ridx[b], semi[b]).wait()
        pltpu.make_async_copy(col2d.at[ch], cidx[b], semi[b]).wait()

    def fire_gathers(b):
        pltpu.async_copy(out_t.at[ridx[b]], abuf[b], semg[b])
        pltpu.async_copy(out_t.at[cidx[b]], bbuf[b], semg[b])

    def wait_gathers(b):
        pltpu.make_async_copy(out_t.at[ridx[b]], abuf[b], semg[b]).wait()
        pltpu.make_async_copy(out_t.at[cidx[b]], bbuf[b], semg[b]).wait()

    def fire_store(ch, b):
        pltpu.async_copy(rbuf[b], res2d.at[ch], semr[b])

    def wait_store(ch, b):
        pltpu.make_async_copy(rbuf[b], res2d.at[ch], semr[b]).wait()

    def compute(b):
        a_, b_ = abuf[b], bbuf[b]

        @pl.loop(0, CHD // 16)
        def _(g):
            for e in range(16):
                r = g * 16 + e
                p = (a_[r, pl.ds(0, 16)] * b_[r, pl.ds(0, 16)]
                     + a_[r, pl.ds(16, 16)] * b_[r, pl.ds(16, 16)])
                q = (a_[r, pl.ds(32, 16)] * b_[r, pl.ds(32, 16)]
                     + a_[r, pl.ds(48, 16)] * b_[r, pl.ds(48, 16)])
                tbuf[e, pl.ds(0, 16)] = p + q
            tot = plsc.load_gather(tbuf, [iota, jnp.full((16,), 0, _i32)])
            for l in range(1, 16):
                tot = tot + plsc.load_gather(
                    tbuf, [iota, jnp.full((16,), l, _i32)])
            rbuf[b][pl.ds(g * 16, 16)] = tot

    fire_idx(c0, 0)
    fire_idx(c0 + 1, 1)
    wait_idx(c0, 0)
    fire_gathers(0)

    @pl.loop(0, DOT_PT // 2)
    def _(k):
        for b in (0, 1):
            j = 2 * k + b
            if b == 0:
                wait_idx(c0 + j + 1, 1)
                wait_gathers(0)
                fire_gathers(1)

                @pl.when(k < DOT_PT // 2 - 1)
                def _():
                    fire_idx(c0 + j + 2, 0)
            else:
                @pl.when(k < DOT_PT // 2 - 1)
                def _():
                    wait_idx(c0 + j + 1, 0)
                    wait_gathers(1)
                    fire_gathers(0)
                    fire_idx(c0 + j + 2, 1)

                @pl.when(k == DOT_PT // 2 - 1)
                def _():
                    wait_gathers(1)

            @pl.when(k >= 1)
            def _():
                wait_store(c0 + j - 2, b)

            compute(b)
            fire_store(c0 + j, b)

    wait_store(c0 + DOT_PT - 2, 0)
    wait_store(c0 + DOT_PT - 1, 1)


def kernel(edge_index, embedding_weight):
    row = edge_index[0]
    col = edge_index[1]
    pad = PAD_E - E
    rowp = jnp.pad(row, (0, pad))
    colp = jnp.pad(col, (0, pad), constant_values=PAD_COL)
    row2dl = rowp.reshape(NCHUNK_L, CHL)
    col2dl = colp.reshape(NCHUNK_L, CHL)
    row2dd = rowp.reshape(NCHUNK_D, CHD)
    col2dd = colp.reshape(NCHUNK_D, CHD)
    x0 = jnp.pad(embedding_weight, ((0, NPAD - N), (0, 0)))

    nd = jax.ShapeDtypeStruct((NPAD, D), _f32)

    k1 = pl.kernel(
        _k1_body,
        out_type=[jax.ShapeDtypeStruct((NPAD,), _f32), nd],
        mesh=_mesh(),
        compiler_params=_CP,
        scratch_types=[
            pltpu.VMEM((NPAD,), _f32),
            pltpu.VMEM_SHARED((NS, NPAD), _f32),
            pltpu.VMEM((TS,), _f32),
            pltpu.VMEM((RCH, D), _f32),
            pltpu.VMEM((CHL,), _i32),
        ])
    dis, y0 = k1(col2dl, x0)

    layer_scratch = [
        pltpu.VMEM_SHARED((ACC_ROWS, D), _f32),
        pltpu.VMEM((CHL, D), _f32),
        pltpu.VMEM((CHL, D), _f32),
        pltpu.VMEM((CHL,), _i32),
        pltpu.VMEM((CHL,), _i32),
        pltpu.VMEM((CHL,), _i32),
        pltpu.VMEM((CHL,), _i32),
        pltpu.VMEM((1, CHL), _i32),
        pltpu.VMEM((1, CHL), _i32),
        pltpu.VMEM((RCH, D), _f32),
        pltpu.VMEM((RCH,), _f32),
        pltpu.SemaphoreType.DMA,
        pltpu.SemaphoreType.DMA,
        pltpu.SemaphoreType.DMA,
        pltpu.SemaphoreType.DMA,
        pltpu.SemaphoreType.DMA,
        pltpu.SemaphoreType.DMA,
    ]
    klayer = pl.kernel(
        functools.partial(_layer_body, False),
        out_type=[nd, nd],
        mesh=_mesh(),
        compiler_params=_CP,
        scratch_types=layer_scratch)
    x1, y1 = klayer(y0, dis, row2dl, col2dl)
    x2, y2 = klayer(y1, dis, row2dl, col2dl)

    kfinal = pl.kernel(
        functools.partial(_layer_body, True),
        out_type=[nd, nd],
        mesh=_mesh(),
        compiler_params=_CP,
        scratch_types=layer_scratch)
    x3, _y3 = kfinal(y2, dis, row2dl, col2dl)

    kcombine = pl.kernel(
        _combine_body,
        out_type=[nd],
        mesh=_mesh(),
        compiler_params=_CP,
        scratch_types=[
            pltpu.VMEM((RCB, D), _f32),
            pltpu.VMEM((RCB, D), _f32),
            pltpu.VMEM((RCB, D), _f32),
            pltpu.VMEM((RCB, D), _f32),
        ])
    (out_t,) = kcombine(x0, x1, x2, x3)

    kdot = pl.kernel(
        _dot_body,
        out_type=[jax.ShapeDtypeStruct((NCHUNK_D, CHD), _f32)],
        mesh=_mesh(),
        compiler_params=_CP,
        scratch_types=[
            pltpu.VMEM((CHD, D), _f32),
            pltpu.VMEM((CHD, D), _f32),
            pltpu.VMEM((CHD, D), _f32),
            pltpu.VMEM((CHD, D), _f32),
            pltpu.VMEM((CHD,), _i32),
            pltpu.VMEM((CHD,), _i32),
            pltpu.VMEM((CHD,), _i32),
            pltpu.VMEM((CHD,), _i32),
            pltpu.VMEM((CHD,), _f32),
            pltpu.VMEM((CHD,), _f32),
            pltpu.VMEM((16, 17), _f32),
            pltpu.SemaphoreType.DMA,
            pltpu.SemaphoreType.DMA,
            pltpu.SemaphoreType.DMA,
            pltpu.SemaphoreType.DMA,
            pltpu.SemaphoreType.DMA,
            pltpu.SemaphoreType.DMA,
        ])
    (res2d,) = kdot(out_t, row2dd, col2dd)
    return res2d.reshape(-1)[:E]


# speedup vs baseline: 12.3859x; 1.0357x over previous
import functools

import jax
import jax.numpy as jnp
from jax import lax
from jax.experimental import pallas as pl
from jax.experimental.pallas import tpu as pltpu
from jax.experimental.pallas import tpu_sc as plsc

N = 50000
D = 64
E = 800000
NUM_LAYERS = 3
ALPHA = 1.0 / (NUM_LAYERS + 1)

NC = 2
NS = 16
NPAD = 51200
TS = NPAD // NS
HALF = NPAD // NC
DUMP = HALF
ACC_ROWS = 25920
CHL = 128
CHD = 256
RCH = 80
CHH = 512
PAD_E = 802816
NCHUNK_L = PAD_E // CHL
NCHUNK_D = PAD_E // CHD
PROP_PT = NCHUNK_L // NS
DOT_PT = NCHUNK_D // (NC * NS)
PAD_COL = NPAD - 1

_i32 = jnp.int32
_f32 = jnp.float32

_CP = pltpu.CompilerParams(needs_layout_passes=False,
                           use_tc_tiling_on_sc=False)


def _mesh():
    return plsc.VectorSubcoreMesh(
        core_axis_name="c", subcore_axis_name="s",
        num_cores=NC, num_subcores=NS)


def _rsqrt16(x):
    i = plsc.bitcast(x, _i32)
    i = jnp.int32(0x5F3759DF) - lax.shift_right_logical(i, 1)
    y = plsc.bitcast(i, _f32)
    for _ in range(3):
        y = y * (1.5 - 0.5 * x * y * y)
    return y


def _k1_body(colh, x0, dis_out, y0_out, hist, stage, wrow, xbuf,
             cbuf0, cbuf1, semi0, semi1):
    c = lax.axis_index("c")
    s = lax.axis_index("s")
    zeros16 = jnp.zeros((16,), _f32)
    ones16 = jnp.ones((16,), _f32)
    iota = lax.iota(_i32, 16)
    cbuf = (cbuf0, cbuf1)
    semi = (semi0, semi1)
    hpt = PAD_E // CHH // NS

    @pl.loop(0, NPAD, step=16)
    def _(j):
        hist[pl.ds(j, 16)] = zeros16

    h0 = s * hpt
    pltpu.async_copy(colh.at[h0], cbuf0, semi0)
    pltpu.async_copy(colh.at[h0 + 1], cbuf1, semi1)

    @pl.loop(0, hpt // 2)
    def _(k):
        for b in (0, 1):
            j = 2 * k + b
            pltpu.make_async_copy(colh.at[h0 + j], cbuf[b], semi[b]).wait()
            for g in range(CHH // 16):
                cc = cbuf[b][pl.ds(g * 16, 16)]
                plsc.addupdate_scatter(hist, [cc], ones16)

            @pl.when(k * 2 + b < hpt - 2)
            def _():
                pltpu.async_copy(colh.at[h0 + j + 2], cbuf[b], semi[b])

    pltpu.sync_copy(hist, stage.at[s])
    plsc.subcore_barrier()

    base = s * TS
    @pl.loop(0, TS, step=16)
    def _(j):
        hist[pl.ds(j, 16)] = zeros16

    for w in range(NS):
        pltpu.sync_copy(stage.at[w, pl.ds(base, TS)], wrow)

        @pl.loop(0, TS, step=16)
        def _(j):
            hist[pl.ds(j, 16)] = hist[pl.ds(j, 16)] + wrow[pl.ds(j, 16)]

    @pl.loop(0, TS, step=16)
    def _(j):
        dv = hist[pl.ds(j, 16)]
        r = _rsqrt16(jnp.maximum(dv, 1.0))
        hist[pl.ds(j, 16)] = jnp.where(dv > 0, r, 0.0)

    @pl.when(c == 0)
    def _():
        pltpu.sync_copy(hist.at[pl.ds(0, TS)], dis_out.at[pl.ds(base, TS)])

    r0 = s * TS + c * (TS // NC)

    @pl.loop(0, TS // NC, step=160)
    def _(g):
        pltpu.sync_copy(x0.at[pl.ds(r0 + g, 160)], xbuf)

        @pl.loop(0, 160, step=16)
        def _(rr):
            dvec = hist[pl.ds(c * (TS // NC) + g + rr, 16)]
            for e in range(16):
                dv = dvec[jnp.full((16,), e, _i32)]
                r = rr + e
                for k in range(D // 16):
                    sl = pl.ds(k * 16, 16)
                    xbuf[r, sl] = xbuf[r, sl] * dv

        pltpu.sync_copy(xbuf, y0_out.at[pl.ds(r0 + g, 160)])


def _layer_body(final, *refs):
    if final:
        (y_prev, dis, row2d, col2d, p012, x_out,
         acc, gbuf0, gbuf1, ridx0, ridx1, cbuf0, cbuf1,
         lidx0, lidx1, ybuf, dbuf,
         semi0, semi1, semg0, semg1, sems0, sems1) = refs
    else:
        (y_prev, dis, row2d, col2d, x_out, y_out,
         acc, gbuf0, gbuf1, ridx0, ridx1, cbuf0, cbuf1,
         lidx0, lidx1, ybuf, dbuf,
         semi0, semi1, semg0, semg1, sems0, sems1) = refs
    c = lax.axis_index("c")
    s = lax.axis_index("s")
    base = c * HALF
    zeros16 = jnp.zeros((16,), _f32)
    neg1 = jnp.full((16,), -1, _i32)
    gbuf = (gbuf0, gbuf1)
    ridx = (ridx0, ridx1)
    cbuf = (cbuf0, cbuf1)
    lidx = (lidx0, lidx1)
    semi = (semi0, semi1)
    semg = (semg0, semg1)
    sems = (sems0, sems1)

    @pl.loop(0, 108)
    def _(r):
        for k in range(D // 16):
            gbuf0[r, pl.ds(k * 16, 16)] = zeros16

    zsrc = gbuf0.at[pl.ds(0, 108)]

    @pl.loop(0, ACC_ROWS // NS, step=108)
    def _(j):
        pltpu.async_copy(
            zsrc, acc.at[pl.ds(s * (ACC_ROWS // NS) + j, 108)], semi0)

    @pl.loop(0, ACC_ROWS // NS, step=108)
    def _(j):
        pltpu.make_async_copy(
            zsrc, acc.at[pl.ds(s * (ACC_ROWS // NS) + j, 108)],
            semi0).wait()

    plsc.subcore_barrier()

    c0 = s * PROP_PT

    def fire_idx(ch, b):
        pltpu.async_copy(row2d.at[ch], ridx[b], semi[b])
        pltpu.async_copy(col2d.at[ch], cbuf[b], semi[b])

    def wait_idx(ch, b):
        pltpu.make_async_copy(row2d.at[ch], ridx[b], semi[b]).wait()
        pltpu.make_async_copy(col2d.at[ch], cbuf[b], semi[b]).wait()

    def fire_gather(b):
        pltpu.async_copy(
            y_prev.at[plsc.Indices(ridx[b], ignored_value=-1)],
            gbuf[b], semg[b])

    def wait_gather(b):
        pltpu.make_async_copy(
            y_prev.at[plsc.Indices(ridx[b], ignored_value=-1)],
            gbuf[b], semg[b]).wait()

    def fire_scatter(b):
        pltpu.async_copy(
            gbuf[b], acc.at[plsc.Indices(lidx[b].at[0], ignored_value=-1)],
            sems[b], add=True)

    def wait_scatter(b):
        pltpu.make_async_copy(
            gbuf[b], acc.at[plsc.Indices(lidx[b].at[0], ignored_value=-1)],
            sems[b]).wait()

    def compute_masks(b):
        for g in range(CHL // 16):
            cc = cbuf[b][pl.ds(g * 16, 16)]
            rv = ridx[b][pl.ds(g * 16, 16)]
            ok = (cc >= base) & (cc < base + HALF)
            ridx[b][pl.ds(g * 16, 16)] = jnp.where(ok, rv, neg1)
            lidx[b][0, pl.ds(g * 16, 16)] = jnp.where(ok, cc - base, neg1)

    fire_idx(c0, 0)
    fire_idx(c0 + 1, 1)

    @pl.loop(0, PROP_PT // 2)
    def _(k):
        for b in (0, 1):
            j = 2 * k + b

            if b == 0:
                @pl.when(k >= 1)
                def _():
                    wait_gather(1)
                    fire_scatter(1)
                    fire_idx(c0 + j + 1, 1)
            else:
                wait_gather(0)
                fire_scatter(0)

                @pl.when(k < PROP_PT // 2 - 1)
                def _():
                    fire_idx(c0 + j + 1, 0)

            wait_idx(c0 + j, b)

            @pl.when(k >= 1)
            def _():
                wait_scatter(b)

            compute_masks(b)
            fire_gather(b)

    wait_gather(1)
    fire_scatter(1)
    wait_scatter(0)
    wait_scatter(1)

    plsc.subcore_barrier()

    r0l = s * (HALF // NS)

    @pl.loop(0, HALF // NS, step=RCH)
    def _(g):
        lr = r0l + g
        gr = base + lr
        wbuf = gbuf0.at[pl.ds(0, RCH)]
        pltpu.sync_copy(acc.at[pl.ds(lr, RCH)], wbuf)
        pltpu.sync_copy(dis.at[pl.ds(gr, RCH)], dbuf)
        if final:
            pltpu.sync_copy(p012.at[pl.ds(gr, RCH)], ybuf)

        @pl.loop(0, RCH, step=16)
        def _(rr):
            dvec = dbuf[pl.ds(rr, 16)]
            for e in range(16):
                dv = dvec[jnp.full((16,), e, _i32)]
                r = rr + e
                for k in range(D // 16):
                    sl = pl.ds(k * 16, 16)
                    xv = wbuf[r, sl] * dv
                    if final:
                        wbuf[r, sl] = ybuf[r, sl] + xv * ALPHA
                    else:
                        wbuf[r, sl] = xv
                        ybuf[r, sl] = xv * dv

        pltpu.sync_copy(wbuf, x_out.at[pl.ds(gr, RCH)])
        if not final:
            pltpu.sync_copy(ybuf, y_out.at[pl.ds(gr, RCH)])


def _p012_body(x0_ref, x1_ref, x2_ref, o_ref):
    o_ref[...] = (x0_ref[...] + x1_ref[...] + x2_ref[...]) * ALPHA


def _dot_body(out_t, row2d, col2d, res2d, abuf0, abuf1, bbuf0, bbuf1,
              ridx0, ridx1, cidx0, cidx1, rbuf0, rbuf1, tbuf,
              semi0, semi1, semg0, semg1, semr0, semr1):
    c = lax.axis_index("c")
    s = lax.axis_index("s")
    wid = s * NC + c
    iota = lax.iota(_i32, 16)
    abuf = (abuf0, abuf1)
    bbuf = (bbuf0, bbuf1)
    ridx = (ridx0, ridx1)
    cidx = (cidx0, cidx1)
    rbuf = (rbuf0, rbuf1)
    semi = (semi0, semi1)
    semg = (semg0, semg1)
    semr = (semr0, semr1)
    c0 = wid * DOT_PT

    def fire_idx(ch, b):
        pltpu.async_copy(row2d.at[ch], ridx[b], semi[b])
        pltpu.async_copy(col2d.at[ch], cidx[b], semi[b])

    def wait_idx(ch, b):
        pltpu.make_async_copy(row2d.at[ch], ridx[b], semi[b]).wait()
        pltpu.make_async_copy(col2d.at[ch], cidx[b], semi[b]).wait()

    def fire_gathers(b):
        pltpu.async_copy(out_t.at[ridx[b]], abuf[b], semg[b])
        pltpu.async_copy(out_t.at[cidx[b]], bbuf[b], semg[b])

    def wait_gathers(b):
        pltpu.make_async_copy(out_t.at[ridx[b]], abuf[b], semg[b]).wait()
        pltpu.make_async_copy(out_t.at[cidx[b]], bbuf[b], semg[b]).wait()

    def fire_store(ch, b):
        pltpu.async_copy(rbuf[b], res2d.at[ch], semr[b])

    def wait_store(ch, b):
        pltpu.make_async_copy(rbuf[b], res2d.at[ch], semr[b]).wait()

    def compute(b):
        a_, b_ = abuf[b], bbuf[b]

        @pl.loop(0, CHD // 16)
        def _(g):
            for e in range(16):
                r = g * 16 + e
                p = (a_[r, pl.ds(0, 16)] * b_[r, pl.ds(0, 16)]
                     + a_[r, pl.ds(16, 16)] * b_[r, pl.ds(16, 16)])
                q = (a_[r, pl.ds(32, 16)] * b_[r, pl.ds(32, 16)]
                     + a_[r, pl.ds(48, 16)] * b_[r, pl.ds(48, 16)])
                tbuf[e, pl.ds(0, 16)] = p + q
            tot = plsc.load_gather(tbuf, [iota, jnp.full((16,), 0, _i32)])
            for l in range(1, 16):
                tot = tot + plsc.load_gather(
                    tbuf, [iota, jnp.full((16,), l, _i32)])
            rbuf[b][pl.ds(g * 16, 16)] = tot

    fire_idx(c0, 0)
    fire_idx(c0 + 1, 1)
    wait_idx(c0, 0)
    fire_gathers(0)

    @pl.loop(0, DOT_PT // 2)
    def _(k):
        for b in (0, 1):
            j = 2 * k + b
            if b == 0:
                wait_idx(c0 + j + 1, 1)
                wait_gathers(0)
                fire_gathers(1)

                @pl.when(k < DOT_PT // 2 - 1)
                def _():
                    fire_idx(c0 + j + 2, 0)
            else:
                @pl.when(k < DOT_PT // 2 - 1)
                def _():
                    wait_idx(c0 + j + 1, 0)
                    wait_gathers(1)
                    fire_gathers(0)
                    fire_idx(c0 + j + 2, 1)

                @pl.when(k == DOT_PT // 2 - 1)
                def _():
                    wait_gathers(1)

            @pl.when(k >= 1)
            def _():
                wait_store(c0 + j - 2, b)

            compute(b)
            fire_store(c0 + j, b)

    wait_store(c0 + DOT_PT - 2, 0)
    wait_store(c0 + DOT_PT - 1, 1)


def kernel(edge_index, embedding_weight):
    row = edge_index[0]
    col = edge_index[1]
    pad = PAD_E - E
    rowp = jnp.pad(row, (0, pad))
    colp = jnp.pad(col, (0, pad), constant_values=PAD_COL)
    row2dl = rowp.reshape(NCHUNK_L, CHL)
    col2dl = colp.reshape(NCHUNK_L, CHL)
    row2dd = rowp.reshape(NCHUNK_D, CHD)
    col2dd = colp.reshape(NCHUNK_D, CHD)
    colh = colp.reshape(PAD_E // CHH, CHH)
    x0 = jnp.pad(embedding_weight, ((0, NPAD - N), (0, 0)))

    nd = jax.ShapeDtypeStruct((NPAD, D), _f32)

    k1 = pl.kernel(
        _k1_body,
        out_type=[jax.ShapeDtypeStruct((NPAD,), _f32), nd],
        mesh=_mesh(),
        compiler_params=_CP,
        scratch_types=[
            pltpu.VMEM((NPAD,), _f32),
            pltpu.VMEM_SHARED((NS, NPAD), _f32),
            pltpu.VMEM((TS,), _f32),
            pltpu.VMEM((160, D), _f32),
            pltpu.VMEM((CHH,), _i32),
            pltpu.VMEM((CHH,), _i32),
            pltpu.SemaphoreType.DMA,
            pltpu.SemaphoreType.DMA,
        ])
    dis, y0 = k1(colh, x0)

    layer_scratch = [
        pltpu.VMEM_SHARED((ACC_ROWS, D), _f32),
        pltpu.VMEM((CHL, D), _f32),
        pltpu.VMEM((CHL, D), _f32),
        pltpu.VMEM((CHL,), _i32),
        pltpu.VMEM((CHL,), _i32),
        pltpu.VMEM((CHL,), _i32),
        pltpu.VMEM((CHL,), _i32),
        pltpu.VMEM((1, CHL), _i32),
        pltpu.VMEM((1, CHL), _i32),
        pltpu.VMEM((RCH, D), _f32),
        pltpu.VMEM((RCH,), _f32),
        pltpu.SemaphoreType.DMA,
        pltpu.SemaphoreType.DMA,
        pltpu.SemaphoreType.DMA,
        pltpu.SemaphoreType.DMA,
        pltpu.SemaphoreType.DMA,
        pltpu.SemaphoreType.DMA,
    ]
    klayer = pl.kernel(
        functools.partial(_layer_body, False),
        out_type=[nd, nd],
        mesh=_mesh(),
        compiler_params=_CP,
        scratch_types=layer_scratch)
    x1, y1 = klayer(y0, dis, row2dl, col2dl)
    x2, y2 = klayer(y1, dis, row2dl, col2dl)

    blk = pl.BlockSpec((1024, D), lambda i: (i, 0))
    p012 = pl.pallas_call(
        _p012_body,
        out_shape=nd,
        grid=(NPAD // 1024,),
        in_specs=[blk, blk, blk],
        out_specs=blk,
    )(x0, x1, x2)

    kfinal = pl.kernel(
        functools.partial(_layer_body, True),
        out_type=[nd],
        mesh=_mesh(),
        compiler_params=_CP,
        scratch_types=layer_scratch)
    (out_t,) = kfinal(y2, dis, row2dl, col2dl, p012)


    kdot = pl.kernel(
        _dot_body,
        out_type=[jax.ShapeDtypeStruct((NCHUNK_D, CHD), _f32)],
        mesh=_mesh(),
        compiler_params=_CP,
        scratch_types=[
            pltpu.VMEM((CHD, D), _f32),
            pltpu.VMEM((CHD, D), _f32),
            pltpu.VMEM((CHD, D), _f32),
            pltpu.VMEM((CHD, D), _f32),
            pltpu.VMEM((CHD,), _i32),
            pltpu.VMEM((CHD,), _i32),
            pltpu.VMEM((CHD,), _i32),
            pltpu.VMEM((CHD,), _i32),
            pltpu.VMEM((CHD,), _f32),
            pltpu.VMEM((CHD,), _f32),
            pltpu.VMEM((16, 17), _f32),
            pltpu.SemaphoreType.DMA,
            pltpu.SemaphoreType.DMA,
            pltpu.SemaphoreType.DMA,
            pltpu.SemaphoreType.DMA,
            pltpu.SemaphoreType.DMA,
            pltpu.SemaphoreType.DMA,
        ])
    (res2d,) = kdot(out_t, row2dd, col2dd)
    return res2d.reshape(-1)[:E]
